# Initial kernel scaffold; baseline (speedup 1.0000x reference)
import jax
import jax.numpy as jnp
from jax.experimental import pallas as pl


def kernel(x, edge_index, edge_attr, log_features, duration, path_length, params):
    raise NotImplementedError("write your pallas kernel here")


# speedup vs baseline: 9.1716x; 9.1716x over previous
import functools
import jax
import jax.numpy as jnp
from jax import lax
from jax.experimental import pallas as pl
from jax.experimental.pallas import tpu as pltpu, tpu_sc as plsc

N = 50000
E = 800000
HID = 64
NC = 2
NS = 16
NW = NC * NS
NHALF = N // 2
CH = 1600
CHUNKS = E // CH
CHS = 256
CHUNKS_S = E // CHS


def _mesh():
  return plsc.VectorSubcoreMesh(core_axis_name="c", subcore_axis_name="s")


_SC_PARAMS = pltpu.CompilerParams(use_tc_tiling_on_sc=False,
                                 needs_layout_passes=False)


def _sc_att_body(src_hbm, dst_hbm, ae_hbm, asrc_hbm, adst_hbm, mvec_hbm,
                 zn_hbm, ex_hbm, den_hbm,
                 asrc_v, adst_v, src_b, dst_b, ae_b, ex_b, mv_v, den_sh):
  cid = lax.axis_index("c")
  sid = lax.axis_index("s")
  wid = sid * NC + cid
  pltpu.sync_copy(asrc_hbm, asrc_v)
  pltpu.sync_copy(adst_hbm, adst_v)
  pltpu.sync_copy(mvec_hbm, mv_v)

  @pl.when(sid == 0)
  def _():
    pltpu.sync_copy(zn_hbm, den_sh)
  plsc.subcore_barrier()

  mv = mv_v[...]
  nk = (CHUNKS - wid + NW - 1) // NW

  def chunk(i, _):
    off = (wid + i * NW) * CH
    pltpu.sync_copy(src_hbm.at[pl.ds(off, CH)], src_b)
    pltpu.sync_copy(dst_hbm.at[pl.ds(off, CH)], dst_b)
    pltpu.sync_copy(ae_hbm.at[pl.ds(off, CH)], ae_b)

    def vec(j, _):
      sl = pl.ds(j * 16, 16)
      si = src_b[sl]
      di = dst_b[sl]
      a = (plsc.load_gather(asrc_v, [si]) + plsc.load_gather(adst_v, [di])
           + ae_b[sl])
      a = jnp.where(a > 0, a, a * jnp.float32(0.2))
      ex_b[sl] = jnp.exp(a - mv)
      return 0
    lax.fori_loop(0, CH // 16, vec, 0, unroll=4)
    pltpu.sync_copy(ex_b, ex_hbm.at[pl.ds(off, CH)])
    pltpu.sync_copy(ex_b, den_sh.at[dst_b], add=True)
    return 0
  lax.fori_loop(0, nk, chunk, 0)

  plsc.subcore_barrier()

  @pl.when(sid == 0)
  def _():
    pltpu.sync_copy(den_sh, den_hbm.at[cid])


def _sc_att(src, dst, ae, asrc, adst, mvec, zn):
  k = pl.kernel(
      _sc_att_body,
      out_type=[jax.ShapeDtypeStruct((E,), jnp.float32),
                jax.ShapeDtypeStruct((NC, N), jnp.float32)],
      mesh=_mesh(),
      compiler_params=_SC_PARAMS,
      scratch_types=[
          pltpu.VMEM((N,), jnp.float32),
          pltpu.VMEM((N,), jnp.float32),
          pltpu.VMEM((CH,), jnp.int32),
          pltpu.VMEM((CH,), jnp.int32),
          pltpu.VMEM((CH,), jnp.float32),
          pltpu.VMEM((CH,), jnp.float32),
          pltpu.VMEM((16,), jnp.float32),
          pltpu.VMEM_SHARED((N,), jnp.float32),
      ],
  )
  return k(src, dst, ae, asrc, adst, mvec, zn)


def _sc_norm_body(dst_hbm, ex_hbm, ew_hbm, den_hbm,
                  w_hbm,
                  den0_v, den1_v, dst_b, ex_b, ew_b, w_b):
  cid = lax.axis_index("c")
  sid = lax.axis_index("s")
  wid = sid * NC + cid
  pltpu.sync_copy(den_hbm.at[0], den0_v)
  pltpu.sync_copy(den_hbm.at[1], den1_v)
  nk = (CHUNKS - wid + NW - 1) // NW

  def chunk(i, _):
    off = (wid + i * NW) * CH
    pltpu.sync_copy(dst_hbm.at[pl.ds(off, CH)], dst_b)
    pltpu.sync_copy(ex_hbm.at[pl.ds(off, CH)], ex_b)
    pltpu.sync_copy(ew_hbm.at[pl.ds(off, CH)], ew_b)

    def vec(j, _):
      sl = pl.ds(j * 16, 16)
      di = dst_b[sl]
      den = plsc.load_gather(den0_v, [di]) + plsc.load_gather(den1_v, [di])
      w_b[sl] = ex_b[sl] / (den + jnp.float32(1e-16)) * ew_b[sl]
      return 0
    lax.fori_loop(0, CH // 16, vec, 0, unroll=4)
    pltpu.sync_copy(w_b, w_hbm.at[pl.ds(off, CH)])
    return 0
  lax.fori_loop(0, nk, chunk, 0)


def _sc_norm(dst, ex, ew, den2):
  k = pl.kernel(
      _sc_norm_body,
      out_type=[jax.ShapeDtypeStruct((E,), jnp.float32)],
      mesh=_mesh(),
      compiler_params=_SC_PARAMS,
      scratch_types=[
          pltpu.VMEM((N,), jnp.float32),
          pltpu.VMEM((N,), jnp.float32),
          pltpu.VMEM((CH,), jnp.int32),
          pltpu.VMEM((CH,), jnp.float32),
          pltpu.VMEM((CH,), jnp.float32),
          pltpu.VMEM((CH,), jnp.float32),
      ],
  )
  return k(dst, ex, ew, den2)[0]


def _sc_spmm_body(src_hbm, dst_hbm, w_hbm, hh_hbm, zacc_hbm,
                  out_hbm,
                  src_b, dst_b, w_b, wm_b, dr_b, rows, sem, acc_sh):
  cid = lax.axis_index("c")
  sid = lax.axis_index("s")
  lo = cid * NHALF

  @pl.when(sid == 0)
  def _():
    pltpu.sync_copy(zacc_hbm, acc_sh)
  plsc.subcore_barrier()

  nk = (CHUNKS_S - sid + NS - 1) // NS

  def chunk(i, _):
    off = (sid + i * NS) * CHS
    pltpu.sync_copy(src_hbm.at[pl.ds(off, CHS)], src_b)
    pltpu.sync_copy(dst_hbm.at[pl.ds(off, CHS)], dst_b)
    pltpu.sync_copy(w_hbm.at[pl.ds(off, CHS)], w_b)

    def vec(j, _):
      sl = pl.ds(j * 16, 16)
      di = dst_b[sl]
      m = (di >= lo) & (di < lo + NHALF)
      wm_b[sl] = jnp.where(m, w_b[sl], jnp.float32(0.0))
      dr_b[sl] = jnp.where(m, di - lo, 0)
      return 0
    lax.fori_loop(0, CHS // 16, vec, 0, unroll=4)

    pltpu.async_copy(hh_hbm.at[src_b], rows, sem).wait()

    def scale(j, _):
      wv = wm_b[pl.ds(j * 16, 16)]
      for r in range(16):
        ws = wv[r]
        row = j * 16 + r
        for q in range(4):
          cs = pl.ds(q * 16, 16)
          rows[row, cs] = rows[row, cs] * ws
      return 0
    lax.fori_loop(0, CHS // 16, scale, 0)

    pltpu.sync_copy(rows, acc_sh.at[dr_b], add=True)
    return 0
  lax.fori_loop(0, nk, chunk, 0)

  plsc.subcore_barrier()

  @pl.when(sid == 0)
  def _():
    pltpu.sync_copy(acc_sh, out_hbm.at[pl.ds(lo, NHALF)])


def _sc_spmm(src, dst, w, hh0, zacc):
  k = pl.kernel(
      _sc_spmm_body,
      out_type=[jax.ShapeDtypeStruct((N, HID), jnp.float32)],
      mesh=_mesh(),
      compiler_params=_SC_PARAMS,
      scratch_types=[
          pltpu.VMEM((CHS,), jnp.int32),
          pltpu.VMEM((CHS,), jnp.int32),
          pltpu.VMEM((CHS,), jnp.float32),
          pltpu.VMEM((CHS,), jnp.float32),
          pltpu.VMEM((CHS,), jnp.int32),
          pltpu.VMEM((CHS, HID), jnp.float32),
          pltpu.SemaphoreType.DMA,
          pltpu.VMEM_SHARED((NHALF, HID), jnp.float32),
      ],
  )
  return k(src, dst, w, hh0, zacc)[0]


def _sc_srcnorm_body(src_hbm, dst_hbm, ex_hbm, ew_hbm, den_hbm, zn_hbm,
                     s_hbm,
                     den0_v, den1_v, src_b, dst_b, ex_b, ew_b, w_b, s_sh):
  cid = lax.axis_index("c")
  sid = lax.axis_index("s")
  wid = sid * NC + cid
  pltpu.sync_copy(den_hbm.at[0], den0_v)
  pltpu.sync_copy(den_hbm.at[1], den1_v)

  @pl.when(sid == 0)
  def _():
    pltpu.sync_copy(zn_hbm, s_sh)
  plsc.subcore_barrier()

  nk = (CHUNKS - wid + NW - 1) // NW

  def chunk(i, _):
    off = (wid + i * NW) * CH
    pltpu.sync_copy(src_hbm.at[pl.ds(off, CH)], src_b)
    pltpu.sync_copy(dst_hbm.at[pl.ds(off, CH)], dst_b)
    pltpu.sync_copy(ex_hbm.at[pl.ds(off, CH)], ex_b)
    pltpu.sync_copy(ew_hbm.at[pl.ds(off, CH)], ew_b)

    def vec(j, _):
      sl = pl.ds(j * 16, 16)
      di = dst_b[sl]
      den = plsc.load_gather(den0_v, [di]) + plsc.load_gather(den1_v, [di])
      w_b[sl] = ex_b[sl] / (den + jnp.float32(1e-16)) * ew_b[sl]
      return 0
    lax.fori_loop(0, CH // 16, vec, 0, unroll=4)
    pltpu.sync_copy(w_b, s_sh.at[src_b], add=True)
    return 0
  lax.fori_loop(0, nk, chunk, 0)

  plsc.subcore_barrier()

  @pl.when(sid == 0)
  def _():
    pltpu.sync_copy(s_sh, s_hbm.at[cid])


def _sc_srcnorm(src, dst, ex, ew, den2, zn):
  k = pl.kernel(
      _sc_srcnorm_body,
      out_type=[jax.ShapeDtypeStruct((NC, N), jnp.float32)],
      mesh=_mesh(),
      compiler_params=_SC_PARAMS,
      scratch_types=[
          pltpu.VMEM((N,), jnp.float32),
          pltpu.VMEM((N,), jnp.float32),
          pltpu.VMEM((CH,), jnp.int32),
          pltpu.VMEM((CH,), jnp.int32),
          pltpu.VMEM((CH,), jnp.float32),
          pltpu.VMEM((CH,), jnp.float32),
          pltpu.VMEM((CH,), jnp.float32),
          pltpu.VMEM_SHARED((N,), jnp.float32),
      ],
  )
  return k(src, dst, ex, ew, den2, zn)[0]


BN = 2000
BE = 8000


def _tc_node_body(x_ref, w1_ref, b1_ref, w2_ref, b2_ref, w_ref, as_ref,
                  ad_ref, hh_ref, asrc_ref, adst_ref, mx_ref):
  h = jnp.maximum(
      jnp.dot(x_ref[...], w1_ref[...], preferred_element_type=jnp.float32)
      + b1_ref[...], 0.0)
  h = jnp.dot(h, w2_ref[...], preferred_element_type=jnp.float32) + b2_ref[...]
  hh = jnp.dot(h, w_ref[...], preferred_element_type=jnp.float32)
  hh_ref[...] = hh
  a_s = jnp.dot(hh, as_ref[...], preferred_element_type=jnp.float32)
  a_d = jnp.dot(hh, ad_ref[...], preferred_element_type=jnp.float32)
  asrc_ref[...] = a_s[:, 0][None, None, :]
  adst_ref[...] = a_d[:, 0][None, None, :]
  mx_ref[...] = jnp.concatenate([jnp.max(a_s, axis=0), jnp.max(a_d, axis=0)
                                 ])[None, None, :]


def _tc_node(x8, w1, b1, w2, b2, w, att_s, att_d):
  g = N // BN
  return pl.pallas_call(
      _tc_node_body,
      grid=(g,),
      in_specs=[
          pl.BlockSpec((BN, 8), lambda i: (i, 0)),
          pl.BlockSpec((8, 32), lambda i: (0, 0)),
          pl.BlockSpec((1, 32), lambda i: (0, 0)),
          pl.BlockSpec((32, HID), lambda i: (0, 0)),
          pl.BlockSpec((1, HID), lambda i: (0, 0)),
          pl.BlockSpec((HID, HID), lambda i: (0, 0)),
          pl.BlockSpec((HID, 1), lambda i: (0, 0)),
          pl.BlockSpec((HID, 1), lambda i: (0, 0)),
      ],
      out_specs=[
          pl.BlockSpec((BN, HID), lambda i: (i, 0)),
          pl.BlockSpec((1, 1, BN), lambda i: (i, 0, 0)),
          pl.BlockSpec((1, 1, BN), lambda i: (i, 0, 0)),
          pl.BlockSpec((1, 1, 2), lambda i: (i, 0, 0)),
      ],
      out_shape=[
          jax.ShapeDtypeStruct((N, HID), jnp.float32),
          jax.ShapeDtypeStruct((g, 1, BN), jnp.float32),
          jax.ShapeDtypeStruct((g, 1, BN), jnp.float32),
          jax.ShapeDtypeStruct((g, 1, 2), jnp.float32),
      ],
  )(x8, w1, b1, w2, b2, w, att_s, att_d)


def _tc_edge_body(ea_ref, w1_ref, b1_ref, w2_ref, b2_ref, w3_ref, b3_ref,
                  v0_ref, v1_ref, ew_ref, ae0_ref, ae1_ref, mx_ref):
  ea = ea_ref[...]
  t = jnp.maximum(
      jnp.dot(ea, w1_ref[...], preferred_element_type=jnp.float32)
      + b1_ref[...], 0.0)
  t = jnp.maximum(
      jnp.dot(t, w2_ref[...], preferred_element_type=jnp.float32)
      + b2_ref[...], 0.0)
  t = jnp.dot(t, w3_ref[...], preferred_element_type=jnp.float32) + b3_ref[...]
  ew_ref[...] = jax.nn.sigmoid(t[:, 0])[None, None, :]
  a0 = jnp.dot(ea, v0_ref[...], preferred_element_type=jnp.float32)
  a1 = jnp.dot(ea, v1_ref[...], preferred_element_type=jnp.float32)
  ae0_ref[...] = a0[:, 0][None, None, :]
  ae1_ref[...] = a1[:, 0][None, None, :]
  mx_ref[...] = jnp.concatenate([jnp.max(a0, axis=0), jnp.max(a1, axis=0)
                                 ])[None, None, :]


def _tc_edge(ea8, w1, b1, w2, b2, w3, b3, v0, v1):
  g = E // BE
  return pl.pallas_call(
      _tc_edge_body,
      grid=(g,),
      in_specs=[
          pl.BlockSpec((BE, 8), lambda i: (i, 0)),
          pl.BlockSpec((8, 32), lambda i: (0, 0)),
          pl.BlockSpec((1, 32), lambda i: (0, 0)),
          pl.BlockSpec((32, 16), lambda i: (0, 0)),
          pl.BlockSpec((1, 16), lambda i: (0, 0)),
          pl.BlockSpec((16, 1), lambda i: (0, 0)),
          pl.BlockSpec((1, 1), lambda i: (0, 0)),
          pl.BlockSpec((8, 1), lambda i: (0, 0)),
          pl.BlockSpec((8, 1), lambda i: (0, 0)),
      ],
      out_specs=[
          pl.BlockSpec((1, 1, BE), lambda i: (i, 0, 0)),
          pl.BlockSpec((1, 1, BE), lambda i: (i, 0, 0)),
          pl.BlockSpec((1, 1, BE), lambda i: (i, 0, 0)),
          pl.BlockSpec((1, 1, 2), lambda i: (i, 0, 0)),
      ],
      out_shape=[
          jax.ShapeDtypeStruct((E // BE, 1, BE), jnp.float32),
          jax.ShapeDtypeStruct((E // BE, 1, BE), jnp.float32),
          jax.ShapeDtypeStruct((E // BE, 1, BE), jnp.float32),
          jax.ShapeDtypeStruct((g, 1, 2), jnp.float32),
      ],
  )(ea8, w1, b1, w2, b2, w3, b3, v0, v1)


def _tc_mid_body(o_ref, b_ref, w_ref, as_ref, ad_ref,
                 hh_ref, asrc_ref, adst_ref, mx_ref):
  v = o_ref[...] + b_ref[...]
  h1 = jnp.where(v > 0, v, jnp.exp(jnp.minimum(v, 0.0)) - 1.0)
  hh = jnp.dot(h1, w_ref[...], preferred_element_type=jnp.float32)
  hh_ref[...] = hh
  a_s = jnp.dot(hh, as_ref[...], preferred_element_type=jnp.float32)
  a_d = jnp.dot(hh, ad_ref[...], preferred_element_type=jnp.float32)
  asrc_ref[...] = a_s[:, 0][None, None, :]
  adst_ref[...] = a_d[:, 0][None, None, :]
  mx_ref[...] = jnp.concatenate([jnp.max(a_s, axis=0), jnp.max(a_d, axis=0)
                                 ])[None, None, :]


def _tc_mid(out0, bias0, w, att_s, att_d):
  g = N // BN
  return pl.pallas_call(
      _tc_mid_body,
      grid=(g,),
      in_specs=[
          pl.BlockSpec((BN, HID), lambda i: (i, 0)),
          pl.BlockSpec((1, HID), lambda i: (0, 0)),
          pl.BlockSpec((HID, HID), lambda i: (0, 0)),
          pl.BlockSpec((HID, 1), lambda i: (0, 0)),
          pl.BlockSpec((HID, 1), lambda i: (0, 0)),
      ],
      out_specs=[
          pl.BlockSpec((BN, HID), lambda i: (i, 0)),
          pl.BlockSpec((1, 1, BN), lambda i: (i, 0, 0)),
          pl.BlockSpec((1, 1, BN), lambda i: (i, 0, 0)),
          pl.BlockSpec((1, 1, 2), lambda i: (i, 0, 0)),
      ],
      out_shape=[
          jax.ShapeDtypeStruct((N, HID), jnp.float32),
          jax.ShapeDtypeStruct((g, 1, BN), jnp.float32),
          jax.ShapeDtypeStruct((g, 1, BN), jnp.float32),
          jax.ShapeDtypeStruct((g, 1, 2), jnp.float32),
      ],
  )(out0, bias0, w, att_s, att_d)


def _tc_matvec_body(s0_ref, s1_ref, hh_ref, o_ref):
  i = pl.program_id(0)

  @pl.when(i == 0)
  def _():
    o_ref[...] = jnp.zeros_like(o_ref)
  sv = (s0_ref[0, 0, :] + s1_ref[0, 0, :])[None, :]
  o_ref[...] += jnp.dot(sv, hh_ref[...], preferred_element_type=jnp.float32)


def _tc_matvec(s2, hh1):
  g = N // BN
  s0 = s2[0].reshape(g, 1, BN)
  s1 = s2[1].reshape(g, 1, BN)
  return pl.pallas_call(
      _tc_matvec_body,
      grid=(g,),
      in_specs=[
          pl.BlockSpec((1, 1, BN), lambda i: (i, 0, 0)),
          pl.BlockSpec((1, 1, BN), lambda i: (i, 0, 0)),
          pl.BlockSpec((BN, HID), lambda i: (i, 0)),
      ],
      out_specs=pl.BlockSpec((1, HID), lambda i: (0, 0)),
      out_shape=jax.ShapeDtypeStruct((1, HID), jnp.float32),
  )(s0, s1, hh1)


def _tc_head_body(hs_ref, b1_ref, lf_ref, dp_ref,
                  r_w, r_b, c1_w, c1_b, c2_w, c2_b, c3_w, c3_b, c4_w, c4_b,
                  f1_w, f1_b, lng, lnb, f2_w, f2_b, k1_w, k1_b, k2_w, k2_b,
                  o_ref):
  hmean = hs_ref[...] * jnp.float32(1.0 / N) + b1_ref[...]
  lf = lf_ref[...].reshape(1, 64)

  def stack3(m):
    z = jnp.zeros((m.shape[0], 1), jnp.float32)
    left = jnp.concatenate([m[:, 1:], z], axis=1)
    right = jnp.concatenate([z, m[:, :-1]], axis=1)
    return jnp.concatenate([right, m, left], axis=0)

  r = jnp.dot(r_w[...], lf, preferred_element_type=jnp.float32) + r_b[...]
  a = jnp.maximum(
      jnp.dot(c1_w[...], stack3(lf), preferred_element_type=jnp.float32)
      + c1_b[...], 0.0)
  a = jnp.maximum(
      jnp.dot(c2_w[...], stack3(a), preferred_element_type=jnp.float32)
      + c2_b[...], 0.0)
  a = jnp.maximum(a + r, 0.0)
  b = jnp.maximum(
      jnp.dot(c3_w[...], stack3(a), preferred_element_type=jnp.float32)
      + c3_b[...], 0.0)
  b = jnp.maximum(
      jnp.dot(c4_w[...], stack3(b), preferred_element_type=jnp.float32)
      + c4_b[...], 0.0)
  a = jnp.maximum(b + a, 0.0)
  lfeat = jnp.mean(a, axis=0)[None, :]

  comb = jnp.concatenate([hmean, dp_ref[...], lfeat], axis=1)
  f = jnp.maximum(
      jnp.dot(comb, f1_w[...], preferred_element_type=jnp.float32)
      + f1_b[...], 0.0)
  mu = jnp.mean(f)
  var = jnp.mean((f - mu) ** 2)
  f = (f - mu) / jnp.sqrt(var + jnp.float32(1e-5)) * lng[...] + lnb[...]
  f = jnp.dot(f, f2_w[...], preferred_element_type=jnp.float32) + f2_b[...]
  c = jnp.maximum(
      jnp.dot(f, k1_w[...], preferred_element_type=jnp.float32)
      + k1_b[...], 0.0)
  o = jnp.dot(c, k2_w[...], preferred_element_type=jnp.float32) + k2_b[...]
  o = o - jnp.max(o)
  o_ref[...] = o - jnp.log(jnp.sum(jnp.exp(o)))


def _tc_head(hsum, bias1, lf, dp, tcn, head):
  full = lambda s: pl.BlockSpec(s, lambda: tuple(0 for _ in s))
  args = [hsum, bias1, lf, dp] + tcn + head
  return pl.pallas_call(
      _tc_head_body,
      in_specs=[full(tuple(a.shape)) for a in args],
      out_specs=full((1, 10)),
      out_shape=jax.ShapeDtypeStruct((1, 10), jnp.float32),
  )(*args)


def kernel(x, edge_index, edge_attr, log_features, duration, path_length,
           params):
  p = params
  f32 = jnp.float32
  src = edge_index[0].astype(jnp.int32)
  dst = edge_index[1].astype(jnp.int32)

  x8 = jnp.pad(x.astype(f32), ((0, 0), (0, 3)))
  ea8 = jnp.pad(edge_attr.astype(f32), ((0, 0), (0, 5)))
  enc_w1 = jnp.pad(p['enc_w1'], ((0, 3), (0, 0)))
  ew_w1 = jnp.pad(p['ew_w1'], ((0, 5), (0, 0)))
  g0, g1 = p['gat0'], p['gat1']
  v0 = jnp.pad(g0['w_edge'] @ g0['att_edge'][0][:, None], ((0, 5), (0, 0)))
  v1 = jnp.pad(g1['w_edge'] @ g1['att_edge'][0][:, None], ((0, 5), (0, 0)))

  zn = jnp.zeros((N,), f32)
  zacc = jnp.zeros((NHALF, HID), f32)

  hh0, asrc0, adst0, mxn = _tc_node(
      x8, enc_w1, p['enc_b1'][None, :], p['enc_w2'], p['enc_b2'][None, :],
      g0['w'], g0['att_src'][0][:, None], g0['att_dst'][0][:, None])
  asrc0, adst0 = asrc0.reshape(N), adst0.reshape(N)
  ew, ae0, ae1, mxe = _tc_edge(
      ea8, ew_w1, p['ew_b1'][None, :], p['ew_w2'], p['ew_b2'][None, :],
      p['ew_w3'], p['ew_b3'][None, :], v0, v1)
  ew, ae0, ae1 = ew.reshape(E), ae0.reshape(E), ae1.reshape(E)

  m0 = jnp.max(mxn[:, 0, 0]) + jnp.max(mxn[:, 0, 1]) + jnp.max(mxe[:, 0, 0])
  m0 = jnp.where(m0 > 0, m0, m0 * f32(0.2))
  mvec0 = jnp.full((16,), m0, f32)

  ex0, den0 = _sc_att(src, dst, ae0, asrc0, adst0, mvec0, zn)
  w0 = _sc_norm(dst, ex0, ew, den0)
  out0 = _sc_spmm(src, dst, w0, hh0, zacc)

  hh1, asrc1, adst1, mxm = _tc_mid(
      out0, g0['bias'][None, :], g1['w'],
      g1['att_src'][0][:, None], g1['att_dst'][0][:, None])
  asrc1, adst1 = asrc1.reshape(N), adst1.reshape(N)
  m1 = jnp.max(mxm[:, 0, 0]) + jnp.max(mxm[:, 0, 1]) + jnp.max(mxe[:, 0, 1])
  m1 = jnp.where(m1 > 0, m1, m1 * f32(0.2))
  mvec1 = jnp.full((16,), m1, f32)

  ex1, den1 = _sc_att(src, dst, ae1, asrc1, adst1, mvec1, zn)
  s2 = _sc_srcnorm(src, dst, ex1, ew, den1, zn)
  hsum = _tc_matvec(s2, hh1)

  dp = jnp.concatenate([duration, path_length]).astype(f32)[None, :]
  def cw(w):
    return jnp.transpose(w, (0, 2, 1)).reshape(w.shape[0], -1)
  tcn = [p['tcn_b1_dw'][:, 0, :], p['tcn_b1_db'][:, None],
         cw(p['tcn_b1_c1_w']), p['tcn_b1_c1_b'][:, None],
         cw(p['tcn_b1_c2_w']), p['tcn_b1_c2_b'][:, None],
         cw(p['tcn_b2_c1_w']), p['tcn_b2_c1_b'][:, None],
         cw(p['tcn_b2_c2_w']), p['tcn_b2_c2_b'][:, None]]
  head = [p['fus_w1'], p['fus_b1'][None, :], p['ln_g'][None, :],
          p['ln_b'][None, :], p['fus_w2'], p['fus_b2'][None, :],
          p['cls_w1'], p['cls_b1'][None, :], p['cls_w2'], p['cls_b2'][None, :]]
  out = _tc_head(hsum, g1['bias'][None, :], log_features, dp, tcn, head)
  return out[0]


# speedup vs baseline: 15.0704x; 1.6432x over previous
import functools
import jax
import jax.numpy as jnp
from jax import lax
from jax.experimental import pallas as pl
from jax.experimental.pallas import tpu as pltpu, tpu_sc as plsc

N = 50000
E = 800000
HID = 64
NC = 2
NS = 16
NW = NC * NS
NHALF = N // 2
CH = 1600
CHUNKS = E // CH
CHS = 256
CHUNKS_S = E // CHS


def _mesh():
  return plsc.VectorSubcoreMesh(core_axis_name="c", subcore_axis_name="s")


_SC_PARAMS = pltpu.CompilerParams(use_tc_tiling_on_sc=False,
                                 needs_layout_passes=False)


def _sc_att_body(src_hbm, dst_hbm, ae_hbm, asrc_hbm, adst_hbm, mvec_hbm,
                 zn_hbm, ex_hbm, den_hbm,
                 asrc_v, adst_v, src_b, dst_b, ae_b, ex_b, mv_v, den_sh):
  cid = lax.axis_index("c")
  sid = lax.axis_index("s")
  wid = sid * NC + cid
  pltpu.sync_copy(asrc_hbm, asrc_v)
  pltpu.sync_copy(adst_hbm, adst_v)
  pltpu.sync_copy(mvec_hbm, mv_v)

  @pl.when(sid == 0)
  def _():
    pltpu.sync_copy(zn_hbm, den_sh)
  plsc.subcore_barrier()

  mv = mv_v[...]
  nk = (CHUNKS - wid + NW - 1) // NW

  def chunk(i, _):
    off = (wid + i * NW) * CH
    pltpu.sync_copy(src_hbm.at[pl.ds(off, CH)], src_b)
    pltpu.sync_copy(dst_hbm.at[pl.ds(off, CH)], dst_b)
    pltpu.sync_copy(ae_hbm.at[pl.ds(off, CH)], ae_b)

    def vec(j, _):
      sl = pl.ds(j * 16, 16)
      si = src_b[sl]
      di = dst_b[sl]
      a = (plsc.load_gather(asrc_v, [si]) + plsc.load_gather(adst_v, [di])
           + ae_b[sl])
      a = jnp.where(a > 0, a, a * jnp.float32(0.2))
      ex_b[sl] = jnp.exp(a - mv)
      return 0
    lax.fori_loop(0, CH // 16, vec, 0, unroll=4)
    pltpu.sync_copy(ex_b, ex_hbm.at[pl.ds(off, CH)])
    pltpu.sync_copy(ex_b, den_sh.at[dst_b], add=True)
    return 0
  lax.fori_loop(0, nk, chunk, 0)

  plsc.subcore_barrier()

  @pl.when(sid == 0)
  def _():
    pltpu.sync_copy(den_sh, den_hbm.at[cid])


def _sc_att(src, dst, ae, asrc, adst, mvec, zn):
  k = pl.kernel(
      _sc_att_body,
      out_type=[jax.ShapeDtypeStruct((E,), jnp.float32),
                jax.ShapeDtypeStruct((NC, N), jnp.float32)],
      mesh=_mesh(),
      compiler_params=_SC_PARAMS,
      scratch_types=[
          pltpu.VMEM((N,), jnp.float32),
          pltpu.VMEM((N,), jnp.float32),
          pltpu.VMEM((CH,), jnp.int32),
          pltpu.VMEM((CH,), jnp.int32),
          pltpu.VMEM((CH,), jnp.float32),
          pltpu.VMEM((CH,), jnp.float32),
          pltpu.VMEM((16,), jnp.float32),
          pltpu.VMEM_SHARED((N,), jnp.float32),
      ],
  )
  return k(src, dst, ae, asrc, adst, mvec, zn)


def _sc_norm_body(dst_hbm, ex_hbm, ew_hbm, den_hbm,
                  w_hbm,
                  den0_v, den1_v, dst_b, ex_b, ew_b, w_b):
  cid = lax.axis_index("c")
  sid = lax.axis_index("s")
  wid = sid * NC + cid
  pltpu.sync_copy(den_hbm.at[0], den0_v)
  pltpu.sync_copy(den_hbm.at[1], den1_v)
  nk = (CHUNKS - wid + NW - 1) // NW

  def chunk(i, _):
    off = (wid + i * NW) * CH
    pltpu.sync_copy(dst_hbm.at[pl.ds(off, CH)], dst_b)
    pltpu.sync_copy(ex_hbm.at[pl.ds(off, CH)], ex_b)
    pltpu.sync_copy(ew_hbm.at[pl.ds(off, CH)], ew_b)

    def vec(j, _):
      sl = pl.ds(j * 16, 16)
      di = dst_b[sl]
      den = plsc.load_gather(den0_v, [di]) + plsc.load_gather(den1_v, [di])
      w_b[sl] = ex_b[sl] / (den + jnp.float32(1e-16)) * ew_b[sl]
      return 0
    lax.fori_loop(0, CH // 16, vec, 0, unroll=4)
    pltpu.sync_copy(w_b, w_hbm.at[pl.ds(off, CH)])
    return 0
  lax.fori_loop(0, nk, chunk, 0)


def _sc_norm(dst, ex, ew, den2):
  k = pl.kernel(
      _sc_norm_body,
      out_type=[jax.ShapeDtypeStruct((E,), jnp.float32)],
      mesh=_mesh(),
      compiler_params=_SC_PARAMS,
      scratch_types=[
          pltpu.VMEM((N,), jnp.float32),
          pltpu.VMEM((N,), jnp.float32),
          pltpu.VMEM((CH,), jnp.int32),
          pltpu.VMEM((CH,), jnp.float32),
          pltpu.VMEM((CH,), jnp.float32),
          pltpu.VMEM((CH,), jnp.float32),
      ],
  )
  return k(dst, ex, ew, den2)[0]


def _sc_spmm_body(src_hbm, dst_hbm, w_hbm, hh_hbm, zacc_hbm,
                  out_hbm,
                  src_b, dst_b, w_b, wm_b, dr_b, rows, sem, acc_sh):
  cid = lax.axis_index("c")
  sid = lax.axis_index("s")
  lo = cid * NHALF

  @pl.when(sid == 0)
  def _():
    pltpu.sync_copy(zacc_hbm, acc_sh)
  plsc.subcore_barrier()

  nk = (CHUNKS_S - sid + NS - 1) // NS

  def chunk(i, _):
    off = (sid + i * NS) * CHS
    pltpu.sync_copy(src_hbm.at[pl.ds(off, CHS)], src_b)
    pltpu.sync_copy(dst_hbm.at[pl.ds(off, CHS)], dst_b)
    pltpu.sync_copy(w_hbm.at[pl.ds(off, CHS)], w_b)

    def vec(j, _):
      sl = pl.ds(j * 16, 16)
      di = dst_b[sl]
      m = (di >= lo) & (di < lo + NHALF)
      wm_b[sl] = jnp.where(m, w_b[sl], jnp.float32(0.0))
      dr_b[sl] = jnp.where(m, di - lo, 0)
      return 0
    lax.fori_loop(0, CHS // 16, vec, 0, unroll=4)

    pltpu.async_copy(hh_hbm.at[src_b], rows, sem).wait()

    def scale(j, _):
      wv = wm_b[pl.ds(j * 16, 16)]
      for r in range(16):
        ws = wv[r]
        row = j * 16 + r
        for q in range(4):
          cs = pl.ds(q * 16, 16)
          rows[row, cs] = rows[row, cs] * ws
      return 0
    lax.fori_loop(0, CHS // 16, scale, 0)

    pltpu.sync_copy(rows, acc_sh.at[dr_b], add=True)
    return 0
  lax.fori_loop(0, nk, chunk, 0)

  plsc.subcore_barrier()

  @pl.when(sid == 0)
  def _():
    pltpu.sync_copy(acc_sh, out_hbm.at[pl.ds(lo, NHALF)])


def _sc_spmm(src, dst, w, hh0, zacc):
  k = pl.kernel(
      _sc_spmm_body,
      out_type=[jax.ShapeDtypeStruct((N, HID), jnp.float32)],
      mesh=_mesh(),
      compiler_params=_SC_PARAMS,
      scratch_types=[
          pltpu.VMEM((CHS,), jnp.int32),
          pltpu.VMEM((CHS,), jnp.int32),
          pltpu.VMEM((CHS,), jnp.float32),
          pltpu.VMEM((CHS,), jnp.float32),
          pltpu.VMEM((CHS,), jnp.int32),
          pltpu.VMEM((CHS, HID), jnp.float32),
          pltpu.SemaphoreType.DMA,
          pltpu.VMEM_SHARED((NHALF, HID), jnp.float32),
      ],
  )
  return k(src, dst, w, hh0, zacc)[0]


def _sc_srcnorm_body(src_hbm, dst_hbm, ex_hbm, ew_hbm, den_hbm, zn_hbm,
                     s_hbm,
                     den0_v, den1_v, src_b, dst_b, ex_b, ew_b, w_b, s_sh):
  cid = lax.axis_index("c")
  sid = lax.axis_index("s")
  wid = sid * NC + cid
  pltpu.sync_copy(den_hbm.at[0], den0_v)
  pltpu.sync_copy(den_hbm.at[1], den1_v)

  @pl.when(sid == 0)
  def _():
    pltpu.sync_copy(zn_hbm, s_sh)
  plsc.subcore_barrier()

  nk = (CHUNKS - wid + NW - 1) // NW

  def chunk(i, _):
    off = (wid + i * NW) * CH
    pltpu.sync_copy(src_hbm.at[pl.ds(off, CH)], src_b)
    pltpu.sync_copy(dst_hbm.at[pl.ds(off, CH)], dst_b)
    pltpu.sync_copy(ex_hbm.at[pl.ds(off, CH)], ex_b)
    pltpu.sync_copy(ew_hbm.at[pl.ds(off, CH)], ew_b)

    def vec(j, _):
      sl = pl.ds(j * 16, 16)
      di = dst_b[sl]
      den = plsc.load_gather(den0_v, [di]) + plsc.load_gather(den1_v, [di])
      w_b[sl] = ex_b[sl] / (den + jnp.float32(1e-16)) * ew_b[sl]
      return 0
    lax.fori_loop(0, CH // 16, vec, 0, unroll=4)
    pltpu.sync_copy(w_b, s_sh.at[src_b], add=True)
    return 0
  lax.fori_loop(0, nk, chunk, 0)

  plsc.subcore_barrier()

  @pl.when(sid == 0)
  def _():
    pltpu.sync_copy(s_sh, s_hbm.at[cid])


def _sc_srcnorm(src, dst, ex, ew, den2, zn):
  k = pl.kernel(
      _sc_srcnorm_body,
      out_type=[jax.ShapeDtypeStruct((NC, N), jnp.float32)],
      mesh=_mesh(),
      compiler_params=_SC_PARAMS,
      scratch_types=[
          pltpu.VMEM((N,), jnp.float32),
          pltpu.VMEM((N,), jnp.float32),
          pltpu.VMEM((CH,), jnp.int32),
          pltpu.VMEM((CH,), jnp.int32),
          pltpu.VMEM((CH,), jnp.float32),
          pltpu.VMEM((CH,), jnp.float32),
          pltpu.VMEM((CH,), jnp.float32),
          pltpu.VMEM_SHARED((N,), jnp.float32),
      ],
  )
  return k(src, dst, ex, ew, den2, zn)[0]


BN = 2000
BE = 8000


def _tc_node_body(x_ref, w1_ref, b1_ref, w2_ref, b2_ref, w_ref, as_ref,
                  ad_ref, hh_ref, asrc_ref, adst_ref, mx_ref):
  h = jnp.maximum(
      jnp.dot(x_ref[...], w1_ref[...], preferred_element_type=jnp.float32)
      + b1_ref[...], 0.0)
  h = jnp.dot(h, w2_ref[...], preferred_element_type=jnp.float32) + b2_ref[...]
  hh = jnp.dot(h, w_ref[...], preferred_element_type=jnp.float32)
  hh_ref[...] = hh
  a_s = jnp.dot(hh, as_ref[...], preferred_element_type=jnp.float32)
  a_d = jnp.dot(hh, ad_ref[...], preferred_element_type=jnp.float32)
  asrc_ref[...] = a_s[:, 0][None, None, :]
  adst_ref[...] = a_d[:, 0][None, None, :]
  mx_ref[...] = jnp.concatenate([jnp.max(a_s, axis=0), jnp.max(a_d, axis=0)
                                 ])[None, None, :]


def _tc_node(x8, w1, b1, w2, b2, w, att_s, att_d):
  g = N // BN
  return pl.pallas_call(
      _tc_node_body,
      grid=(g,),
      in_specs=[
          pl.BlockSpec((BN, 5), lambda i: (i, 0)),
          pl.BlockSpec((5, 32), lambda i: (0, 0)),
          pl.BlockSpec((1, 32), lambda i: (0, 0)),
          pl.BlockSpec((32, HID), lambda i: (0, 0)),
          pl.BlockSpec((1, HID), lambda i: (0, 0)),
          pl.BlockSpec((HID, HID), lambda i: (0, 0)),
          pl.BlockSpec((HID, 1), lambda i: (0, 0)),
          pl.BlockSpec((HID, 1), lambda i: (0, 0)),
      ],
      out_specs=[
          pl.BlockSpec((BN, HID), lambda i: (i, 0)),
          pl.BlockSpec((1, 1, BN), lambda i: (i, 0, 0)),
          pl.BlockSpec((1, 1, BN), lambda i: (i, 0, 0)),
          pl.BlockSpec((1, 1, 2), lambda i: (i, 0, 0)),
      ],
      out_shape=[
          jax.ShapeDtypeStruct((N, HID), jnp.float32),
          jax.ShapeDtypeStruct((g, 1, BN), jnp.float32),
          jax.ShapeDtypeStruct((g, 1, BN), jnp.float32),
          jax.ShapeDtypeStruct((g, 1, 2), jnp.float32),
      ],
  )(x8, w1, b1, w2, b2, w, att_s, att_d)


def _tc_edge_body(ea_ref, w1_ref, b1_ref, w2_ref, b2_ref, w3_ref, b3_ref,
                  v0_ref, v1_ref, ew_ref, ae0_ref, ae1_ref, mx_ref):
  ea = ea_ref[...]
  t = jnp.maximum(
      jnp.dot(ea, w1_ref[...], preferred_element_type=jnp.float32)
      + b1_ref[...], 0.0)
  t = jnp.maximum(
      jnp.dot(t, w2_ref[...], preferred_element_type=jnp.float32)
      + b2_ref[...], 0.0)
  t = jnp.dot(t, w3_ref[...], preferred_element_type=jnp.float32) + b3_ref[...]
  ew_ref[...] = jax.nn.sigmoid(t[:, 0])[None, None, :]
  a0 = jnp.dot(ea, v0_ref[...], preferred_element_type=jnp.float32)
  a1 = jnp.dot(ea, v1_ref[...], preferred_element_type=jnp.float32)
  ae0_ref[...] = a0[:, 0][None, None, :]
  ae1_ref[...] = a1[:, 0][None, None, :]
  mx_ref[...] = jnp.concatenate([jnp.max(a0, axis=0), jnp.max(a1, axis=0)
                                 ])[None, None, :]


def _tc_edge(ea8, w1, b1, w2, b2, w3, b3, v0, v1):
  g = E // BE
  return pl.pallas_call(
      _tc_edge_body,
      grid=(g,),
      in_specs=[
          pl.BlockSpec((BE, 3), lambda i: (i, 0)),
          pl.BlockSpec((3, 32), lambda i: (0, 0)),
          pl.BlockSpec((1, 32), lambda i: (0, 0)),
          pl.BlockSpec((32, 16), lambda i: (0, 0)),
          pl.BlockSpec((1, 16), lambda i: (0, 0)),
          pl.BlockSpec((16, 1), lambda i: (0, 0)),
          pl.BlockSpec((1, 1), lambda i: (0, 0)),
          pl.BlockSpec((3, 1), lambda i: (0, 0)),
          pl.BlockSpec((3, 1), lambda i: (0, 0)),
      ],
      out_specs=[
          pl.BlockSpec((1, 1, BE), lambda i: (i, 0, 0)),
          pl.BlockSpec((1, 1, BE), lambda i: (i, 0, 0)),
          pl.BlockSpec((1, 1, BE), lambda i: (i, 0, 0)),
          pl.BlockSpec((1, 1, 2), lambda i: (i, 0, 0)),
      ],
      out_shape=[
          jax.ShapeDtypeStruct((E // BE, 1, BE), jnp.float32),
          jax.ShapeDtypeStruct((E // BE, 1, BE), jnp.float32),
          jax.ShapeDtypeStruct((E // BE, 1, BE), jnp.float32),
          jax.ShapeDtypeStruct((g, 1, 2), jnp.float32),
      ],
  )(ea8, w1, b1, w2, b2, w3, b3, v0, v1)


def _tc_mid_body(o_ref, b_ref, w_ref, as_ref, ad_ref,
                 hh_ref, asrc_ref, adst_ref, mx_ref):
  v = o_ref[...] + b_ref[...]
  h1 = jnp.where(v > 0, v, jnp.exp(jnp.minimum(v, 0.0)) - 1.0)
  hh = jnp.dot(h1, w_ref[...], preferred_element_type=jnp.float32)
  hh_ref[...] = hh
  a_s = jnp.dot(hh, as_ref[...], preferred_element_type=jnp.float32)
  a_d = jnp.dot(hh, ad_ref[...], preferred_element_type=jnp.float32)
  asrc_ref[...] = a_s[:, 0][None, None, :]
  adst_ref[...] = a_d[:, 0][None, None, :]
  mx_ref[...] = jnp.concatenate([jnp.max(a_s, axis=0), jnp.max(a_d, axis=0)
                                 ])[None, None, :]


def _tc_mid(out0, bias0, w, att_s, att_d):
  g = N // BN
  return pl.pallas_call(
      _tc_mid_body,
      grid=(g,),
      in_specs=[
          pl.BlockSpec((BN, HID), lambda i: (i, 0)),
          pl.BlockSpec((1, HID), lambda i: (0, 0)),
          pl.BlockSpec((HID, HID), lambda i: (0, 0)),
          pl.BlockSpec((HID, 1), lambda i: (0, 0)),
          pl.BlockSpec((HID, 1), lambda i: (0, 0)),
      ],
      out_specs=[
          pl.BlockSpec((BN, HID), lambda i: (i, 0)),
          pl.BlockSpec((1, 1, BN), lambda i: (i, 0, 0)),
          pl.BlockSpec((1, 1, BN), lambda i: (i, 0, 0)),
          pl.BlockSpec((1, 1, 2), lambda i: (i, 0, 0)),
      ],
      out_shape=[
          jax.ShapeDtypeStruct((N, HID), jnp.float32),
          jax.ShapeDtypeStruct((g, 1, BN), jnp.float32),
          jax.ShapeDtypeStruct((g, 1, BN), jnp.float32),
          jax.ShapeDtypeStruct((g, 1, 2), jnp.float32),
      ],
  )(out0, bias0, w, att_s, att_d)


def _tc_matvec_body(s0_ref, s1_ref, hh_ref, o_ref):
  i = pl.program_id(0)

  @pl.when(i == 0)
  def _():
    o_ref[...] = jnp.zeros_like(o_ref)
  sv = (s0_ref[0, 0, :] + s1_ref[0, 0, :])[None, :]
  o_ref[...] += jnp.dot(sv, hh_ref[...], preferred_element_type=jnp.float32)


def _tc_matvec(s2, hh1):
  g = N // BN
  s0 = s2[0].reshape(g, 1, BN)
  s1 = s2[1].reshape(g, 1, BN)
  return pl.pallas_call(
      _tc_matvec_body,
      grid=(g,),
      in_specs=[
          pl.BlockSpec((1, 1, BN), lambda i: (i, 0, 0)),
          pl.BlockSpec((1, 1, BN), lambda i: (i, 0, 0)),
          pl.BlockSpec((BN, HID), lambda i: (i, 0)),
      ],
      out_specs=pl.BlockSpec((1, HID), lambda i: (0, 0)),
      out_shape=jax.ShapeDtypeStruct((1, HID), jnp.float32),
  )(s0, s1, hh1)


def _tc_head_body(hs_ref, b1_ref, lf_ref, dp_ref,
                  r_w, r_b, c1_w, c1_b, c2_w, c2_b, c3_w, c3_b, c4_w, c4_b,
                  f1_w, f1_b, lng, lnb, f2_w, f2_b, k1_w, k1_b, k2_w, k2_b,
                  o_ref):
  hmean = hs_ref[...] * jnp.float32(1.0 / N) + b1_ref[...]
  lf = lf_ref[...].reshape(1, 64)

  def stack3(m):
    z = jnp.zeros((m.shape[0], 1), jnp.float32)
    left = jnp.concatenate([m[:, 1:], z], axis=1)
    right = jnp.concatenate([z, m[:, :-1]], axis=1)
    return jnp.concatenate([right, m, left], axis=0)

  r = jnp.dot(r_w[...], lf, preferred_element_type=jnp.float32) + r_b[...]
  a = jnp.maximum(
      jnp.dot(c1_w[...], stack3(lf), preferred_element_type=jnp.float32)
      + c1_b[...], 0.0)
  a = jnp.maximum(
      jnp.dot(c2_w[...], stack3(a), preferred_element_type=jnp.float32)
      + c2_b[...], 0.0)
  a = jnp.maximum(a + r, 0.0)
  b = jnp.maximum(
      jnp.dot(c3_w[...], stack3(a), preferred_element_type=jnp.float32)
      + c3_b[...], 0.0)
  b = jnp.maximum(
      jnp.dot(c4_w[...], stack3(b), preferred_element_type=jnp.float32)
      + c4_b[...], 0.0)
  a = jnp.maximum(b + a, 0.0)
  lfeat = jnp.mean(a, axis=0)[None, :]

  comb = jnp.concatenate([hmean, dp_ref[...], lfeat], axis=1)
  f = jnp.maximum(
      jnp.dot(comb, f1_w[...], preferred_element_type=jnp.float32)
      + f1_b[...], 0.0)
  mu = jnp.mean(f)
  var = jnp.mean((f - mu) ** 2)
  f = (f - mu) / jnp.sqrt(var + jnp.float32(1e-5)) * lng[...] + lnb[...]
  f = jnp.dot(f, f2_w[...], preferred_element_type=jnp.float32) + f2_b[...]
  c = jnp.maximum(
      jnp.dot(f, k1_w[...], preferred_element_type=jnp.float32)
      + k1_b[...], 0.0)
  o = jnp.dot(c, k2_w[...], preferred_element_type=jnp.float32) + k2_b[...]
  o = o - jnp.max(o)
  o_ref[...] = o - jnp.log(jnp.sum(jnp.exp(o)))


def _tc_head(hsum, bias1, lf, dp, tcn, head):
  full = lambda s: pl.BlockSpec(s, lambda: tuple(0 for _ in s))
  args = [hsum, bias1, lf, dp] + tcn + head
  return pl.pallas_call(
      _tc_head_body,
      in_specs=[full(tuple(a.shape)) for a in args],
      out_specs=full((1, 10)),
      out_shape=jax.ShapeDtypeStruct((1, 10), jnp.float32),
  )(*args)


def kernel(x, edge_index, edge_attr, log_features, duration, path_length,
           params):
  p = params
  f32 = jnp.float32
  src = edge_index[0].astype(jnp.int32)
  dst = edge_index[1].astype(jnp.int32)

  x8 = x.astype(f32)
  ea8 = edge_attr.astype(f32)
  enc_w1 = p['enc_w1']
  ew_w1 = p['ew_w1']
  g0, g1 = p['gat0'], p['gat1']
  v0 = g0['w_edge'] @ g0['att_edge'][0][:, None]
  v1 = g1['w_edge'] @ g1['att_edge'][0][:, None]

  zn = jnp.zeros((N,), f32)
  zacc = jnp.zeros((NHALF, HID), f32)

  hh0, asrc0, adst0, mxn = _tc_node(
      x8, enc_w1, p['enc_b1'][None, :], p['enc_w2'], p['enc_b2'][None, :],
      g0['w'], g0['att_src'][0][:, None], g0['att_dst'][0][:, None])
  asrc0, adst0 = asrc0.reshape(N), adst0.reshape(N)
  ew, ae0, ae1, mxe = _tc_edge(
      ea8, ew_w1, p['ew_b1'][None, :], p['ew_w2'], p['ew_b2'][None, :],
      p['ew_w3'], p['ew_b3'][None, :], v0, v1)
  ew, ae0, ae1 = ew.reshape(E), ae0.reshape(E), ae1.reshape(E)

  m0 = jnp.max(mxn[:, 0, 0]) + jnp.max(mxn[:, 0, 1]) + jnp.max(mxe[:, 0, 0])
  m0 = jnp.where(m0 > 0, m0, m0 * f32(0.2))
  mvec0 = jnp.full((16,), m0, f32)

  ex0, den0 = _sc_att(src, dst, ae0, asrc0, adst0, mvec0, zn)
  w0 = _sc_norm(dst, ex0, ew, den0)
  out0 = _sc_spmm(src, dst, w0, hh0, zacc)

  hh1, asrc1, adst1, mxm = _tc_mid(
      out0, g0['bias'][None, :], g1['w'],
      g1['att_src'][0][:, None], g1['att_dst'][0][:, None])
  asrc1, adst1 = asrc1.reshape(N), adst1.reshape(N)
  m1 = jnp.max(mxm[:, 0, 0]) + jnp.max(mxm[:, 0, 1]) + jnp.max(mxe[:, 0, 1])
  m1 = jnp.where(m1 > 0, m1, m1 * f32(0.2))
  mvec1 = jnp.full((16,), m1, f32)

  ex1, den1 = _sc_att(src, dst, ae1, asrc1, adst1, mvec1, zn)
  s2 = _sc_srcnorm(src, dst, ex1, ew, den1, zn)
  hsum = _tc_matvec(s2, hh1)

  dp = jnp.concatenate([duration, path_length]).astype(f32)[None, :]
  def cw(w):
    return jnp.transpose(w, (0, 2, 1)).reshape(w.shape[0], -1)
  tcn = [p['tcn_b1_dw'][:, 0, :], p['tcn_b1_db'][:, None],
         cw(p['tcn_b1_c1_w']), p['tcn_b1_c1_b'][:, None],
         cw(p['tcn_b1_c2_w']), p['tcn_b1_c2_b'][:, None],
         cw(p['tcn_b2_c1_w']), p['tcn_b2_c1_b'][:, None],
         cw(p['tcn_b2_c2_w']), p['tcn_b2_c2_b'][:, None]]
  head = [p['fus_w1'], p['fus_b1'][None, :], p['ln_g'][None, :],
          p['ln_b'][None, :], p['fus_w2'], p['fus_b2'][None, :],
          p['cls_w1'], p['cls_b1'][None, :], p['cls_w2'], p['cls_b2'][None, :]]
  out = _tc_head(hsum, g1['bias'][None, :], log_features, dp, tcn, head)
  return out[0]


# speedup vs baseline: 15.4330x; 1.0241x over previous
import functools
import jax
import jax.numpy as jnp
from jax import lax
from jax.experimental import pallas as pl
from jax.experimental.pallas import tpu as pltpu, tpu_sc as plsc

N = 50000
E = 800000
HID = 64
NC = 2
NS = 16
NW = NC * NS
NHALF = N // 2
CH = 1600
CHUNKS = E // CH
CHS = 160
CHUNKS_S = E // CHS


def _mesh():
  return plsc.VectorSubcoreMesh(core_axis_name="c", subcore_axis_name="s")


_SC_PARAMS = pltpu.CompilerParams(use_tc_tiling_on_sc=False,
                                 needs_layout_passes=False)


def _sc_att_body(src_hbm, dst_hbm, ae_hbm, asrc_hbm, adst_hbm, mvec_hbm,
                 zn_hbm, ex_hbm, den_hbm,
                 asrc_v, adst_v, src_b, dst_b, ae_b, ex_b, mv_v, den_sh):
  cid = lax.axis_index("c")
  sid = lax.axis_index("s")
  wid = sid * NC + cid
  pltpu.sync_copy(asrc_hbm, asrc_v)
  pltpu.sync_copy(adst_hbm, adst_v)
  pltpu.sync_copy(mvec_hbm, mv_v)

  @pl.when(sid == 0)
  def _():
    pltpu.sync_copy(zn_hbm, den_sh)
  plsc.subcore_barrier()

  mv = mv_v[...]
  nk = (CHUNKS - wid + NW - 1) // NW

  def chunk(i, _):
    off = (wid + i * NW) * CH
    pltpu.sync_copy(src_hbm.at[pl.ds(off, CH)], src_b)
    pltpu.sync_copy(dst_hbm.at[pl.ds(off, CH)], dst_b)
    pltpu.sync_copy(ae_hbm.at[pl.ds(off, CH)], ae_b)

    def vec(j, _):
      sl = pl.ds(j * 16, 16)
      si = src_b[sl]
      di = dst_b[sl]
      a = (plsc.load_gather(asrc_v, [si]) + plsc.load_gather(adst_v, [di])
           + ae_b[sl])
      a = jnp.where(a > 0, a, a * jnp.float32(0.2))
      ex_b[sl] = jnp.exp(a - mv)
      return 0
    lax.fori_loop(0, CH // 16, vec, 0, unroll=4)
    pltpu.sync_copy(ex_b, ex_hbm.at[pl.ds(off, CH)])
    pltpu.sync_copy(ex_b, den_sh.at[dst_b], add=True)
    return 0
  lax.fori_loop(0, nk, chunk, 0)

  plsc.subcore_barrier()

  @pl.when(sid == 0)
  def _():
    pltpu.sync_copy(den_sh, den_hbm.at[cid])


def _sc_att(src, dst, ae, asrc, adst, mvec, zn):
  k = pl.kernel(
      _sc_att_body,
      out_type=[jax.ShapeDtypeStruct((E,), jnp.float32),
                jax.ShapeDtypeStruct((NC, N), jnp.float32)],
      mesh=_mesh(),
      compiler_params=_SC_PARAMS,
      scratch_types=[
          pltpu.VMEM((N,), jnp.float32),
          pltpu.VMEM((N,), jnp.float32),
          pltpu.VMEM((CH,), jnp.int32),
          pltpu.VMEM((CH,), jnp.int32),
          pltpu.VMEM((CH,), jnp.float32),
          pltpu.VMEM((CH,), jnp.float32),
          pltpu.VMEM((16,), jnp.float32),
          pltpu.VMEM_SHARED((N,), jnp.float32),
      ],
  )
  return k(src, dst, ae, asrc, adst, mvec, zn)


def _sc_norm_body(dst_hbm, ex_hbm, ew_hbm, den_hbm,
                  w_hbm,
                  den0_v, den1_v, dst_b, ex_b, ew_b, w_b):
  cid = lax.axis_index("c")
  sid = lax.axis_index("s")
  wid = sid * NC + cid
  pltpu.sync_copy(den_hbm.at[0], den0_v)
  pltpu.sync_copy(den_hbm.at[1], den1_v)
  nk = (CHUNKS - wid + NW - 1) // NW

  def chunk(i, _):
    off = (wid + i * NW) * CH
    pltpu.sync_copy(dst_hbm.at[pl.ds(off, CH)], dst_b)
    pltpu.sync_copy(ex_hbm.at[pl.ds(off, CH)], ex_b)
    pltpu.sync_copy(ew_hbm.at[pl.ds(off, CH)], ew_b)

    def vec(j, _):
      sl = pl.ds(j * 16, 16)
      di = dst_b[sl]
      den = plsc.load_gather(den0_v, [di]) + plsc.load_gather(den1_v, [di])
      w_b[sl] = ex_b[sl] / (den + jnp.float32(1e-16)) * ew_b[sl]
      return 0
    lax.fori_loop(0, CH // 16, vec, 0, unroll=4)
    pltpu.sync_copy(w_b, w_hbm.at[pl.ds(off, CH)])
    return 0
  lax.fori_loop(0, nk, chunk, 0)


def _sc_norm(dst, ex, ew, den2):
  k = pl.kernel(
      _sc_norm_body,
      out_type=[jax.ShapeDtypeStruct((E,), jnp.float32)],
      mesh=_mesh(),
      compiler_params=_SC_PARAMS,
      scratch_types=[
          pltpu.VMEM((N,), jnp.float32),
          pltpu.VMEM((N,), jnp.float32),
          pltpu.VMEM((CH,), jnp.int32),
          pltpu.VMEM((CH,), jnp.float32),
          pltpu.VMEM((CH,), jnp.float32),
          pltpu.VMEM((CH,), jnp.float32),
      ],
  )
  return k(dst, ex, ew, den2)[0]


def _sc_spmm_body(src_hbm, dst_hbm, w_hbm, hh_hbm, zacc_hbm,
                  out_hbm,
                  src_a, dst_a, w_a, wm_a, dr_a, rows_a, sem_a,
                  src_c, dst_c, w_c, wm_c, dr_c, rows_c, sem_c, acc_sh):
  cid = lax.axis_index("c")
  sid = lax.axis_index("s")
  lo = cid * NHALF

  @pl.when(sid == 0)
  def _():
    pltpu.sync_copy(zacc_hbm, acc_sh)
  plsc.subcore_barrier()

  nk = (CHUNKS_S - sid + NS - 1) // NS

  def load_and_fire(c, bufs):
    src_b, dst_b, w_b, wm_b, dr_b, rows, sem = bufs
    off = (sid + c * NS) * CHS
    pltpu.sync_copy(src_hbm.at[pl.ds(off, CHS)], src_b)
    pltpu.sync_copy(dst_hbm.at[pl.ds(off, CHS)], dst_b)
    pltpu.sync_copy(w_hbm.at[pl.ds(off, CHS)], w_b)

    def vec(j, _):
      sl = pl.ds(j * 16, 16)
      di = dst_b[sl]
      m = (di >= lo) & (di < lo + NHALF)
      wm_b[sl] = jnp.where(m, w_b[sl], jnp.float32(0.0))
      dr_b[sl] = jnp.where(m, di - lo, 0)
      return 0
    lax.fori_loop(0, CHS // 16, vec, 0, unroll=4)
    return pltpu.async_copy(hh_hbm.at[src_b], rows, sem)

  def drain(desc, bufs):
    _, _, _, wm_b, dr_b, rows, _ = bufs
    desc.wait()

    def scale(j, _):
      wv = wm_b[pl.ds(j * 16, 16)]
      for r in range(16):
        ws = wv[r]
        row = j * 16 + r
        for q in range(4):
          cs = pl.ds(q * 16, 16)
          rows[row, cs] = rows[row, cs] * ws
      return 0
    lax.fori_loop(0, CHS // 16, scale, 0)
    pltpu.sync_copy(rows, acc_sh.at[dr_b], add=True)

  bufs_a = (src_a, dst_a, w_a, wm_a, dr_a, rows_a, sem_a)
  bufs_c = (src_c, dst_c, w_c, wm_c, dr_c, rows_c, sem_c)

  def pair(t, _):
    d0 = load_and_fire(2 * t, bufs_a)
    d1 = load_and_fire(2 * t + 1, bufs_c)
    drain(d0, bufs_a)
    drain(d1, bufs_c)
    return 0
  lax.fori_loop(0, nk // 2, pair, 0)

  @pl.when(nk % 2 == 1)
  def _():
    drain(load_and_fire(nk - 1, bufs_a), bufs_a)

  plsc.subcore_barrier()

  @pl.when(sid == 0)
  def _():
    pltpu.sync_copy(acc_sh, out_hbm.at[pl.ds(lo, NHALF)])


def _sc_spmm(src, dst, w, hh0, zacc):
  buf = lambda: [pltpu.VMEM((CHS,), jnp.int32),
                 pltpu.VMEM((CHS,), jnp.int32),
                 pltpu.VMEM((CHS,), jnp.float32),
                 pltpu.VMEM((CHS,), jnp.float32),
                 pltpu.VMEM((CHS,), jnp.int32),
                 pltpu.VMEM((CHS, HID), jnp.float32),
                 pltpu.SemaphoreType.DMA]
  k = pl.kernel(
      _sc_spmm_body,
      out_type=[jax.ShapeDtypeStruct((N, HID), jnp.float32)],
      mesh=_mesh(),
      compiler_params=_SC_PARAMS,
      scratch_types=buf() + buf() + [
          pltpu.VMEM_SHARED((NHALF, HID), jnp.float32),
      ],
  )
  return k(src, dst, w, hh0, zacc)[0]


def _sc_srcnorm_body(src_hbm, dst_hbm, ex_hbm, ew_hbm, den_hbm, zn_hbm,
                     s_hbm,
                     den0_v, den1_v, src_b, dst_b, ex_b, ew_b, w_b, s_sh):
  cid = lax.axis_index("c")
  sid = lax.axis_index("s")
  wid = sid * NC + cid
  pltpu.sync_copy(den_hbm.at[0], den0_v)
  pltpu.sync_copy(den_hbm.at[1], den1_v)

  @pl.when(sid == 0)
  def _():
    pltpu.sync_copy(zn_hbm, s_sh)
  plsc.subcore_barrier()

  nk = (CHUNKS - wid + NW - 1) // NW

  def chunk(i, _):
    off = (wid + i * NW) * CH
    pltpu.sync_copy(src_hbm.at[pl.ds(off, CH)], src_b)
    pltpu.sync_copy(dst_hbm.at[pl.ds(off, CH)], dst_b)
    pltpu.sync_copy(ex_hbm.at[pl.ds(off, CH)], ex_b)
    pltpu.sync_copy(ew_hbm.at[pl.ds(off, CH)], ew_b)

    def vec(j, _):
      sl = pl.ds(j * 16, 16)
      di = dst_b[sl]
      den = plsc.load_gather(den0_v, [di]) + plsc.load_gather(den1_v, [di])
      w_b[sl] = ex_b[sl] / (den + jnp.float32(1e-16)) * ew_b[sl]
      return 0
    lax.fori_loop(0, CH // 16, vec, 0, unroll=4)
    pltpu.sync_copy(w_b, s_sh.at[src_b], add=True)
    return 0
  lax.fori_loop(0, nk, chunk, 0)

  plsc.subcore_barrier()

  @pl.when(sid == 0)
  def _():
    pltpu.sync_copy(s_sh, s_hbm.at[cid])


def _sc_srcnorm(src, dst, ex, ew, den2, zn):
  k = pl.kernel(
      _sc_srcnorm_body,
      out_type=[jax.ShapeDtypeStruct((NC, N), jnp.float32)],
      mesh=_mesh(),
      compiler_params=_SC_PARAMS,
      scratch_types=[
          pltpu.VMEM((N,), jnp.float32),
          pltpu.VMEM((N,), jnp.float32),
          pltpu.VMEM((CH,), jnp.int32),
          pltpu.VMEM((CH,), jnp.int32),
          pltpu.VMEM((CH,), jnp.float32),
          pltpu.VMEM((CH,), jnp.float32),
          pltpu.VMEM((CH,), jnp.float32),
          pltpu.VMEM_SHARED((N,), jnp.float32),
      ],
  )
  return k(src, dst, ex, ew, den2, zn)[0]


BN = 2000
BE = 8000


def _tc_node_body(x_ref, w1_ref, b1_ref, w2_ref, b2_ref, w_ref, as_ref,
                  ad_ref, hh_ref, asrc_ref, adst_ref, mx_ref):
  h = jnp.maximum(
      jnp.dot(x_ref[...], w1_ref[...], preferred_element_type=jnp.float32)
      + b1_ref[...], 0.0)
  h = jnp.dot(h, w2_ref[...], preferred_element_type=jnp.float32) + b2_ref[...]
  hh = jnp.dot(h, w_ref[...], preferred_element_type=jnp.float32)
  hh_ref[...] = hh
  a_s = jnp.dot(hh, as_ref[...], preferred_element_type=jnp.float32)
  a_d = jnp.dot(hh, ad_ref[...], preferred_element_type=jnp.float32)
  asrc_ref[...] = a_s[:, 0][None, None, :]
  adst_ref[...] = a_d[:, 0][None, None, :]
  mx_ref[...] = jnp.concatenate([jnp.max(a_s, axis=0), jnp.max(a_d, axis=0)
                                 ])[None, None, :]


def _tc_node(x8, w1, b1, w2, b2, w, att_s, att_d):
  g = N // BN
  return pl.pallas_call(
      _tc_node_body,
      grid=(g,),
      in_specs=[
          pl.BlockSpec((BN, 5), lambda i: (i, 0)),
          pl.BlockSpec((5, 32), lambda i: (0, 0)),
          pl.BlockSpec((1, 32), lambda i: (0, 0)),
          pl.BlockSpec((32, HID), lambda i: (0, 0)),
          pl.BlockSpec((1, HID), lambda i: (0, 0)),
          pl.BlockSpec((HID, HID), lambda i: (0, 0)),
          pl.BlockSpec((HID, 1), lambda i: (0, 0)),
          pl.BlockSpec((HID, 1), lambda i: (0, 0)),
      ],
      out_specs=[
          pl.BlockSpec((BN, HID), lambda i: (i, 0)),
          pl.BlockSpec((1, 1, BN), lambda i: (i, 0, 0)),
          pl.BlockSpec((1, 1, BN), lambda i: (i, 0, 0)),
          pl.BlockSpec((1, 1, 2), lambda i: (i, 0, 0)),
      ],
      out_shape=[
          jax.ShapeDtypeStruct((N, HID), jnp.float32),
          jax.ShapeDtypeStruct((g, 1, BN), jnp.float32),
          jax.ShapeDtypeStruct((g, 1, BN), jnp.float32),
          jax.ShapeDtypeStruct((g, 1, 2), jnp.float32),
      ],
  )(x8, w1, b1, w2, b2, w, att_s, att_d)


def _tc_edge_body(ea_ref, w1_ref, b1_ref, w2_ref, b2_ref, w3_ref, b3_ref,
                  v0_ref, v1_ref, ew_ref, ae0_ref, ae1_ref, mx_ref):
  ea = ea_ref[...]
  t = jnp.maximum(
      jnp.dot(ea, w1_ref[...], preferred_element_type=jnp.float32)
      + b1_ref[...], 0.0)
  t = jnp.maximum(
      jnp.dot(t, w2_ref[...], preferred_element_type=jnp.float32)
      + b2_ref[...], 0.0)
  t = jnp.dot(t, w3_ref[...], preferred_element_type=jnp.float32) + b3_ref[...]
  ew_ref[...] = jax.nn.sigmoid(t[:, 0])[None, None, :]
  a0 = jnp.dot(ea, v0_ref[...], preferred_element_type=jnp.float32)
  a1 = jnp.dot(ea, v1_ref[...], preferred_element_type=jnp.float32)
  ae0_ref[...] = a0[:, 0][None, None, :]
  ae1_ref[...] = a1[:, 0][None, None, :]
  mx_ref[...] = jnp.concatenate([jnp.max(a0, axis=0), jnp.max(a1, axis=0)
                                 ])[None, None, :]


def _tc_edge(ea8, w1, b1, w2, b2, w3, b3, v0, v1):
  g = E // BE
  return pl.pallas_call(
      _tc_edge_body,
      grid=(g,),
      in_specs=[
          pl.BlockSpec((BE, 3), lambda i: (i, 0)),
          pl.BlockSpec((3, 32), lambda i: (0, 0)),
          pl.BlockSpec((1, 32), lambda i: (0, 0)),
          pl.BlockSpec((32, 16), lambda i: (0, 0)),
          pl.BlockSpec((1, 16), lambda i: (0, 0)),
          pl.BlockSpec((16, 1), lambda i: (0, 0)),
          pl.BlockSpec((1, 1), lambda i: (0, 0)),
          pl.BlockSpec((3, 1), lambda i: (0, 0)),
          pl.BlockSpec((3, 1), lambda i: (0, 0)),
      ],
      out_specs=[
          pl.BlockSpec((1, 1, BE), lambda i: (i, 0, 0)),
          pl.BlockSpec((1, 1, BE), lambda i: (i, 0, 0)),
          pl.BlockSpec((1, 1, BE), lambda i: (i, 0, 0)),
          pl.BlockSpec((1, 1, 2), lambda i: (i, 0, 0)),
      ],
      out_shape=[
          jax.ShapeDtypeStruct((E // BE, 1, BE), jnp.float32),
          jax.ShapeDtypeStruct((E // BE, 1, BE), jnp.float32),
          jax.ShapeDtypeStruct((E // BE, 1, BE), jnp.float32),
          jax.ShapeDtypeStruct((g, 1, 2), jnp.float32),
      ],
  )(ea8, w1, b1, w2, b2, w3, b3, v0, v1)


def _tc_mid_body(o_ref, b_ref, w_ref, as_ref, ad_ref,
                 hh_ref, asrc_ref, adst_ref, mx_ref):
  v = o_ref[...] + b_ref[...]
  h1 = jnp.where(v > 0, v, jnp.exp(jnp.minimum(v, 0.0)) - 1.0)
  hh = jnp.dot(h1, w_ref[...], preferred_element_type=jnp.float32)
  hh_ref[...] = hh
  a_s = jnp.dot(hh, as_ref[...], preferred_element_type=jnp.float32)
  a_d = jnp.dot(hh, ad_ref[...], preferred_element_type=jnp.float32)
  asrc_ref[...] = a_s[:, 0][None, None, :]
  adst_ref[...] = a_d[:, 0][None, None, :]
  mx_ref[...] = jnp.concatenate([jnp.max(a_s, axis=0), jnp.max(a_d, axis=0)
                                 ])[None, None, :]


def _tc_mid(out0, bias0, w, att_s, att_d):
  g = N // BN
  return pl.pallas_call(
      _tc_mid_body,
      grid=(g,),
      in_specs=[
          pl.BlockSpec((BN, HID), lambda i: (i, 0)),
          pl.BlockSpec((1, HID), lambda i: (0, 0)),
          pl.BlockSpec((HID, HID), lambda i: (0, 0)),
          pl.BlockSpec((HID, 1), lambda i: (0, 0)),
          pl.BlockSpec((HID, 1), lambda i: (0, 0)),
      ],
      out_specs=[
          pl.BlockSpec((BN, HID), lambda i: (i, 0)),
          pl.BlockSpec((1, 1, BN), lambda i: (i, 0, 0)),
          pl.BlockSpec((1, 1, BN), lambda i: (i, 0, 0)),
          pl.BlockSpec((1, 1, 2), lambda i: (i, 0, 0)),
      ],
      out_shape=[
          jax.ShapeDtypeStruct((N, HID), jnp.float32),
          jax.ShapeDtypeStruct((g, 1, BN), jnp.float32),
          jax.ShapeDtypeStruct((g, 1, BN), jnp.float32),
          jax.ShapeDtypeStruct((g, 1, 2), jnp.float32),
      ],
  )(out0, bias0, w, att_s, att_d)


def _tc_matvec_body(s0_ref, s1_ref, hh_ref, o_ref):
  i = pl.program_id(0)

  @pl.when(i == 0)
  def _():
    o_ref[...] = jnp.zeros_like(o_ref)
  sv = (s0_ref[0, 0, :] + s1_ref[0, 0, :])[None, :]
  o_ref[...] += jnp.dot(sv, hh_ref[...], preferred_element_type=jnp.float32)


def _tc_matvec(s2, hh1):
  g = N // BN
  s0 = s2[0].reshape(g, 1, BN)
  s1 = s2[1].reshape(g, 1, BN)
  return pl.pallas_call(
      _tc_matvec_body,
      grid=(g,),
      in_specs=[
          pl.BlockSpec((1, 1, BN), lambda i: (i, 0, 0)),
          pl.BlockSpec((1, 1, BN), lambda i: (i, 0, 0)),
          pl.BlockSpec((BN, HID), lambda i: (i, 0)),
      ],
      out_specs=pl.BlockSpec((1, HID), lambda i: (0, 0)),
      out_shape=jax.ShapeDtypeStruct((1, HID), jnp.float32),
  )(s0, s1, hh1)


def _tc_head_body(hs_ref, b1_ref, lf_ref, dp_ref,
                  r_w, r_b, c1_w, c1_b, c2_w, c2_b, c3_w, c3_b, c4_w, c4_b,
                  f1_w, f1_b, lng, lnb, f2_w, f2_b, k1_w, k1_b, k2_w, k2_b,
                  o_ref):
  hmean = hs_ref[...] * jnp.float32(1.0 / N) + b1_ref[...]
  lf = lf_ref[...].reshape(1, 64)

  def stack3(m):
    z = jnp.zeros((m.shape[0], 1), jnp.float32)
    left = jnp.concatenate([m[:, 1:], z], axis=1)
    right = jnp.concatenate([z, m[:, :-1]], axis=1)
    return jnp.concatenate([right, m, left], axis=0)

  r = jnp.dot(r_w[...], lf, preferred_element_type=jnp.float32) + r_b[...]
  a = jnp.maximum(
      jnp.dot(c1_w[...], stack3(lf), preferred_element_type=jnp.float32)
      + c1_b[...], 0.0)
  a = jnp.maximum(
      jnp.dot(c2_w[...], stack3(a), preferred_element_type=jnp.float32)
      + c2_b[...], 0.0)
  a = jnp.maximum(a + r, 0.0)
  b = jnp.maximum(
      jnp.dot(c3_w[...], stack3(a), preferred_element_type=jnp.float32)
      + c3_b[...], 0.0)
  b = jnp.maximum(
      jnp.dot(c4_w[...], stack3(b), preferred_element_type=jnp.float32)
      + c4_b[...], 0.0)
  a = jnp.maximum(b + a, 0.0)
  lfeat = jnp.mean(a, axis=0)[None, :]

  comb = jnp.concatenate([hmean, dp_ref[...], lfeat], axis=1)
  f = jnp.maximum(
      jnp.dot(comb, f1_w[...], preferred_element_type=jnp.float32)
      + f1_b[...], 0.0)
  mu = jnp.mean(f)
  var = jnp.mean((f - mu) ** 2)
  f = (f - mu) / jnp.sqrt(var + jnp.float32(1e-5)) * lng[...] + lnb[...]
  f = jnp.dot(f, f2_w[...], preferred_element_type=jnp.float32) + f2_b[...]
  c = jnp.maximum(
      jnp.dot(f, k1_w[...], preferred_element_type=jnp.float32)
      + k1_b[...], 0.0)
  o = jnp.dot(c, k2_w[...], preferred_element_type=jnp.float32) + k2_b[...]
  o = o - jnp.max(o)
  o_ref[...] = o - jnp.log(jnp.sum(jnp.exp(o)))


def _tc_head(hsum, bias1, lf, dp, tcn, head):
  full = lambda s: pl.BlockSpec(s, lambda: tuple(0 for _ in s))
  args = [hsum, bias1, lf, dp] + tcn + head
  return pl.pallas_call(
      _tc_head_body,
      in_specs=[full(tuple(a.shape)) for a in args],
      out_specs=full((1, 10)),
      out_shape=jax.ShapeDtypeStruct((1, 10), jnp.float32),
  )(*args)


def kernel(x, edge_index, edge_attr, log_features, duration, path_length,
           params):
  p = params
  f32 = jnp.float32
  src = edge_index[0].astype(jnp.int32)
  dst = edge_index[1].astype(jnp.int32)

  x8 = x.astype(f32)
  ea8 = edge_attr.astype(f32)
  enc_w1 = p['enc_w1']
  ew_w1 = p['ew_w1']
  g0, g1 = p['gat0'], p['gat1']
  v0 = g0['w_edge'] @ g0['att_edge'][0][:, None]
  v1 = g1['w_edge'] @ g1['att_edge'][0][:, None]

  zn = jnp.zeros((N,), f32)
  zacc = jnp.zeros((NHALF, HID), f32)

  hh0, asrc0, adst0, mxn = _tc_node(
      x8, enc_w1, p['enc_b1'][None, :], p['enc_w2'], p['enc_b2'][None, :],
      g0['w'], g0['att_src'][0][:, None], g0['att_dst'][0][:, None])
  asrc0, adst0 = asrc0.reshape(N), adst0.reshape(N)
  ew, ae0, ae1, mxe = _tc_edge(
      ea8, ew_w1, p['ew_b1'][None, :], p['ew_w2'], p['ew_b2'][None, :],
      p['ew_w3'], p['ew_b3'][None, :], v0, v1)
  ew, ae0, ae1 = ew.reshape(E), ae0.reshape(E), ae1.reshape(E)

  m0 = jnp.max(mxn[:, 0, 0]) + jnp.max(mxn[:, 0, 1]) + jnp.max(mxe[:, 0, 0])
  m0 = jnp.where(m0 > 0, m0, m0 * f32(0.2))
  mvec0 = jnp.full((16,), m0, f32)

  ex0, den0 = _sc_att(src, dst, ae0, asrc0, adst0, mvec0, zn)
  w0 = _sc_norm(dst, ex0, ew, den0)
  out0 = _sc_spmm(src, dst, w0, hh0, zacc)

  hh1, asrc1, adst1, mxm = _tc_mid(
      out0, g0['bias'][None, :], g1['w'],
      g1['att_src'][0][:, None], g1['att_dst'][0][:, None])
  asrc1, adst1 = asrc1.reshape(N), adst1.reshape(N)
  m1 = jnp.max(mxm[:, 0, 0]) + jnp.max(mxm[:, 0, 1]) + jnp.max(mxe[:, 0, 1])
  m1 = jnp.where(m1 > 0, m1, m1 * f32(0.2))
  mvec1 = jnp.full((16,), m1, f32)

  ex1, den1 = _sc_att(src, dst, ae1, asrc1, adst1, mvec1, zn)
  s2 = _sc_srcnorm(src, dst, ex1, ew, den1, zn)
  hsum = _tc_matvec(s2, hh1)

  dp = jnp.concatenate([duration, path_length]).astype(f32)[None, :]
  def cw(w):
    return jnp.transpose(w, (0, 2, 1)).reshape(w.shape[0], -1)
  tcn = [p['tcn_b1_dw'][:, 0, :], p['tcn_b1_db'][:, None],
         cw(p['tcn_b1_c1_w']), p['tcn_b1_c1_b'][:, None],
         cw(p['tcn_b1_c2_w']), p['tcn_b1_c2_b'][:, None],
         cw(p['tcn_b2_c1_w']), p['tcn_b2_c1_b'][:, None],
         cw(p['tcn_b2_c2_w']), p['tcn_b2_c2_b'][:, None]]
  head = [p['fus_w1'], p['fus_b1'][None, :], p['ln_g'][None, :],
          p['ln_b'][None, :], p['fus_w2'], p['fus_b2'][None, :],
          p['cls_w1'], p['cls_b1'][None, :], p['cls_w2'], p['cls_b2'][None, :]]
  out = _tc_head(hsum, g1['bias'][None, :], log_features, dp, tcn, head)
  return out[0]


# speedup vs baseline: 16.4293x; 1.0646x over previous
import functools
import jax
import jax.numpy as jnp
from jax import lax
from jax.experimental import pallas as pl
from jax.experimental.pallas import tpu as pltpu, tpu_sc as plsc

N = 50000
E = 800000
HID = 64
NC = 2
NS = 16
NW = NC * NS
NHALF = N // 2
CH = 1600
CHUNKS = E // CH
CHS = 160
CHUNKS_S = E // CHS


def _mesh():
  return plsc.VectorSubcoreMesh(core_axis_name="c", subcore_axis_name="s")


_SC_PARAMS = pltpu.CompilerParams(use_tc_tiling_on_sc=False,
                                 needs_layout_passes=False)


def _sc_att_body(src_hbm, dst_hbm, ae_hbm, asrc_hbm, adst_hbm, mvec_hbm,
                 zn_hbm, ex_hbm, den_hbm,
                 asrc_v, adst_v, src_b, dst_b, ae_b, ex_b, mv_v, den_sh):
  cid = lax.axis_index("c")
  sid = lax.axis_index("s")
  wid = sid * NC + cid
  pltpu.sync_copy(asrc_hbm, asrc_v)
  pltpu.sync_copy(adst_hbm, adst_v)
  pltpu.sync_copy(mvec_hbm, mv_v)

  @pl.when(sid == 0)
  def _():
    pltpu.sync_copy(zn_hbm, den_sh)
  plsc.subcore_barrier()

  mv = mv_v[...]
  nk = (CHUNKS - wid + NW - 1) // NW

  def chunk(i, _):
    off = (wid + i * NW) * CH
    pltpu.sync_copy(src_hbm.at[pl.ds(off, CH)], src_b)
    pltpu.sync_copy(dst_hbm.at[pl.ds(off, CH)], dst_b)
    pltpu.sync_copy(ae_hbm.at[pl.ds(off, CH)], ae_b)

    def vec(j, _):
      sl = pl.ds(j * 16, 16)
      si = src_b[sl]
      di = dst_b[sl]
      a = (plsc.load_gather(asrc_v, [si]) + plsc.load_gather(adst_v, [di])
           + ae_b[sl])
      a = jnp.where(a > 0, a, a * jnp.float32(0.2))
      ex_b[sl] = jnp.exp(a - mv)
      return 0
    lax.fori_loop(0, CH // 16, vec, 0, unroll=4)
    pltpu.sync_copy(ex_b, ex_hbm.at[pl.ds(off, CH)])
    pltpu.sync_copy(ex_b, den_sh.at[dst_b], add=True)
    return 0
  lax.fori_loop(0, nk, chunk, 0)

  plsc.subcore_barrier()

  @pl.when(sid == 0)
  def _():
    pltpu.sync_copy(den_sh, den_hbm.at[cid])


def _sc_att(src, dst, ae, asrc, adst, mvec, zn):
  k = pl.kernel(
      _sc_att_body,
      out_type=[jax.ShapeDtypeStruct((E,), jnp.float32),
                jax.ShapeDtypeStruct((NC, N), jnp.float32)],
      mesh=_mesh(),
      compiler_params=_SC_PARAMS,
      scratch_types=[
          pltpu.VMEM((N,), jnp.float32),
          pltpu.VMEM((N,), jnp.float32),
          pltpu.VMEM((CH,), jnp.int32),
          pltpu.VMEM((CH,), jnp.int32),
          pltpu.VMEM((CH,), jnp.float32),
          pltpu.VMEM((CH,), jnp.float32),
          pltpu.VMEM((16,), jnp.float32),
          pltpu.VMEM_SHARED((N,), jnp.float32),
      ],
  )
  return k(src, dst, ae, asrc, adst, mvec, zn)


def _sc_norm_body(dst_hbm, ex_hbm, ew_hbm, den_hbm,
                  w_hbm,
                  den0_v, den1_v, dst_b, ex_b, ew_b, w_b):
  cid = lax.axis_index("c")
  sid = lax.axis_index("s")
  wid = sid * NC + cid
  pltpu.sync_copy(den_hbm.at[0], den0_v)
  pltpu.sync_copy(den_hbm.at[1], den1_v)
  nk = (CHUNKS - wid + NW - 1) // NW

  def chunk(i, _):
    off = (wid + i * NW) * CH
    pltpu.sync_copy(dst_hbm.at[pl.ds(off, CH)], dst_b)
    pltpu.sync_copy(ex_hbm.at[pl.ds(off, CH)], ex_b)
    pltpu.sync_copy(ew_hbm.at[pl.ds(off, CH)], ew_b)

    def vec(j, _):
      sl = pl.ds(j * 16, 16)
      di = dst_b[sl]
      den = plsc.load_gather(den0_v, [di]) + plsc.load_gather(den1_v, [di])
      w_b[sl] = ex_b[sl] / (den + jnp.float32(1e-16)) * ew_b[sl]
      return 0
    lax.fori_loop(0, CH // 16, vec, 0, unroll=4)
    pltpu.sync_copy(w_b, w_hbm.at[pl.ds(off, CH)])
    return 0
  lax.fori_loop(0, nk, chunk, 0)


def _sc_norm(dst, ex, ew, den2):
  k = pl.kernel(
      _sc_norm_body,
      out_type=[jax.ShapeDtypeStruct((E,), jnp.float32)],
      mesh=_mesh(),
      compiler_params=_SC_PARAMS,
      scratch_types=[
          pltpu.VMEM((N,), jnp.float32),
          pltpu.VMEM((N,), jnp.float32),
          pltpu.VMEM((CH,), jnp.int32),
          pltpu.VMEM((CH,), jnp.float32),
          pltpu.VMEM((CH,), jnp.float32),
          pltpu.VMEM((CH,), jnp.float32),
      ],
  )
  return k(dst, ex, ew, den2)[0]


def _sc_spmm_body(src_hbm, dst_hbm, w_hbm, hh_hbm, zacc_hbm,
                  out_hbm,
                  src_a, dst_a, w_a, wm_a, dr_a, rows_a, sem_a,
                  src_c, dst_c, w_c, wm_c, dr_c, rows_c, sem_c, acc_sh):
  cid = lax.axis_index("c")
  sid = lax.axis_index("s")
  lo = cid * NHALF

  @pl.when(sid == 0)
  def _():
    pltpu.sync_copy(zacc_hbm, acc_sh)
  plsc.subcore_barrier()

  nk = (CHUNKS_S - sid + NS - 1) // NS

  def load_and_fire(c, bufs):
    src_b, dst_b, w_b, wm_b, dr_b, rows, sem = bufs
    off = (sid + c * NS) * CHS
    d1 = pltpu.async_copy(src_hbm.at[pl.ds(off, CHS)], src_b, sem)
    d2 = pltpu.async_copy(dst_hbm.at[pl.ds(off, CHS)], dst_b, sem)
    d3 = pltpu.async_copy(w_hbm.at[pl.ds(off, CHS)], w_b, sem)
    d1.wait(); d2.wait(); d3.wait()

    def vec(j, _):
      sl = pl.ds(j * 16, 16)
      di = dst_b[sl]
      m = (di >= lo) & (di < lo + NHALF)
      wm_b[sl] = jnp.where(m, w_b[sl], jnp.float32(0.0))
      dr_b[sl] = jnp.where(m, di - lo, 0)
      return 0
    lax.fori_loop(0, CHS // 16, vec, 0, unroll=4)
    return pltpu.async_copy(hh_hbm.at[src_b], rows, sem)

  def drain(desc, bufs):
    _, _, _, wm_b, dr_b, rows, _ = bufs
    desc.wait()

    def scale(j, _):
      wv = wm_b[pl.ds(j * 16, 16)]
      for r in range(16):
        ws = wv[r]
        row = j * 16 + r
        for q in range(4):
          cs = pl.ds(q * 16, 16)
          rows[row, cs] = rows[row, cs] * ws
      return 0
    lax.fori_loop(0, CHS // 16, scale, 0)
    pltpu.sync_copy(rows, acc_sh.at[dr_b], add=True)

  bufs_a = (src_a, dst_a, w_a, wm_a, dr_a, rows_a, sem_a)
  bufs_c = (src_c, dst_c, w_c, wm_c, dr_c, rows_c, sem_c)

  def pair(t, _):
    d0 = load_and_fire(2 * t, bufs_a)
    d1 = load_and_fire(2 * t + 1, bufs_c)
    drain(d0, bufs_a)
    drain(d1, bufs_c)
    return 0
  lax.fori_loop(0, nk // 2, pair, 0)

  @pl.when(nk % 2 == 1)
  def _():
    drain(load_and_fire(nk - 1, bufs_a), bufs_a)

  plsc.subcore_barrier()

  @pl.when(sid == 0)
  def _():
    pltpu.sync_copy(acc_sh, out_hbm.at[pl.ds(lo, NHALF)])


def _sc_spmm(src, dst, w, hh0, zacc):
  buf = lambda: [pltpu.VMEM((CHS,), jnp.int32),
                 pltpu.VMEM((CHS,), jnp.int32),
                 pltpu.VMEM((CHS,), jnp.float32),
                 pltpu.VMEM((CHS,), jnp.float32),
                 pltpu.VMEM((CHS,), jnp.int32),
                 pltpu.VMEM((CHS, HID), jnp.float32),
                 pltpu.SemaphoreType.DMA]
  k = pl.kernel(
      _sc_spmm_body,
      out_type=[jax.ShapeDtypeStruct((N, HID), jnp.float32)],
      mesh=_mesh(),
      compiler_params=_SC_PARAMS,
      scratch_types=buf() + buf() + [
          pltpu.VMEM_SHARED((NHALF, HID), jnp.float32),
      ],
  )
  return k(src, dst, w, hh0, zacc)[0]


def _sc_srcnorm_body(src_hbm, dst_hbm, ex_hbm, ew_hbm, den_hbm, zn_hbm,
                     s_hbm,
                     den0_v, den1_v, src_b, dst_b, ex_b, ew_b, w_b, s_sh):
  cid = lax.axis_index("c")
  sid = lax.axis_index("s")
  wid = sid * NC + cid
  pltpu.sync_copy(den_hbm.at[0], den0_v)
  pltpu.sync_copy(den_hbm.at[1], den1_v)

  @pl.when(sid == 0)
  def _():
    pltpu.sync_copy(zn_hbm, s_sh)
  plsc.subcore_barrier()

  nk = (CHUNKS - wid + NW - 1) // NW

  def chunk(i, _):
    off = (wid + i * NW) * CH
    pltpu.sync_copy(src_hbm.at[pl.ds(off, CH)], src_b)
    pltpu.sync_copy(dst_hbm.at[pl.ds(off, CH)], dst_b)
    pltpu.sync_copy(ex_hbm.at[pl.ds(off, CH)], ex_b)
    pltpu.sync_copy(ew_hbm.at[pl.ds(off, CH)], ew_b)

    def vec(j, _):
      sl = pl.ds(j * 16, 16)
      di = dst_b[sl]
      den = plsc.load_gather(den0_v, [di]) + plsc.load_gather(den1_v, [di])
      w_b[sl] = ex_b[sl] / (den + jnp.float32(1e-16)) * ew_b[sl]
      return 0
    lax.fori_loop(0, CH // 16, vec, 0, unroll=4)
    pltpu.sync_copy(w_b, s_sh.at[src_b], add=True)
    return 0
  lax.fori_loop(0, nk, chunk, 0)

  plsc.subcore_barrier()

  @pl.when(sid == 0)
  def _():
    pltpu.sync_copy(s_sh, s_hbm.at[cid])


def _sc_srcnorm(src, dst, ex, ew, den2, zn):
  k = pl.kernel(
      _sc_srcnorm_body,
      out_type=[jax.ShapeDtypeStruct((NC, N), jnp.float32)],
      mesh=_mesh(),
      compiler_params=_SC_PARAMS,
      scratch_types=[
          pltpu.VMEM((N,), jnp.float32),
          pltpu.VMEM((N,), jnp.float32),
          pltpu.VMEM((CH,), jnp.int32),
          pltpu.VMEM((CH,), jnp.int32),
          pltpu.VMEM((CH,), jnp.float32),
          pltpu.VMEM((CH,), jnp.float32),
          pltpu.VMEM((CH,), jnp.float32),
          pltpu.VMEM_SHARED((N,), jnp.float32),
      ],
  )
  return k(src, dst, ex, ew, den2, zn)[0]


BN = 2000
BE = 8000


def _tc_node_body(x_ref, w1_ref, b1_ref, w2_ref, b2_ref, w_ref, as_ref,
                  ad_ref, hh_ref, asrc_ref, adst_ref, mx_ref):
  h = jnp.maximum(
      jnp.dot(x_ref[...], w1_ref[...], preferred_element_type=jnp.float32)
      + b1_ref[...], 0.0)
  h = jnp.dot(h, w2_ref[...], preferred_element_type=jnp.float32) + b2_ref[...]
  hh = jnp.dot(h, w_ref[...], preferred_element_type=jnp.float32)
  hh_ref[...] = hh
  a_s = jnp.dot(hh, as_ref[...], preferred_element_type=jnp.float32)
  a_d = jnp.dot(hh, ad_ref[...], preferred_element_type=jnp.float32)
  asrc_ref[...] = a_s[:, 0][None, None, :]
  adst_ref[...] = a_d[:, 0][None, None, :]
  mx_ref[...] = jnp.concatenate([jnp.max(a_s, axis=0), jnp.max(a_d, axis=0)
                                 ])[None, None, :]


def _tc_node(x8, w1, b1, w2, b2, w, att_s, att_d):
  g = N // BN
  return pl.pallas_call(
      _tc_node_body,
      grid=(g,),
      in_specs=[
          pl.BlockSpec((BN, 5), lambda i: (i, 0)),
          pl.BlockSpec((5, 32), lambda i: (0, 0)),
          pl.BlockSpec((1, 32), lambda i: (0, 0)),
          pl.BlockSpec((32, HID), lambda i: (0, 0)),
          pl.BlockSpec((1, HID), lambda i: (0, 0)),
          pl.BlockSpec((HID, HID), lambda i: (0, 0)),
          pl.BlockSpec((HID, 1), lambda i: (0, 0)),
          pl.BlockSpec((HID, 1), lambda i: (0, 0)),
      ],
      out_specs=[
          pl.BlockSpec((BN, HID), lambda i: (i, 0)),
          pl.BlockSpec((1, 1, BN), lambda i: (i, 0, 0)),
          pl.BlockSpec((1, 1, BN), lambda i: (i, 0, 0)),
          pl.BlockSpec((1, 1, 2), lambda i: (i, 0, 0)),
      ],
      out_shape=[
          jax.ShapeDtypeStruct((N, HID), jnp.float32),
          jax.ShapeDtypeStruct((g, 1, BN), jnp.float32),
          jax.ShapeDtypeStruct((g, 1, BN), jnp.float32),
          jax.ShapeDtypeStruct((g, 1, 2), jnp.float32),
      ],
  )(x8, w1, b1, w2, b2, w, att_s, att_d)


def _tc_edge_body(ea_ref, w1_ref, b1_ref, w2_ref, b2_ref, w3_ref, b3_ref,
                  v0_ref, v1_ref, ew_ref, ae0_ref, ae1_ref, mx_ref):
  ea = ea_ref[...]
  t = jnp.maximum(
      jnp.dot(ea, w1_ref[...], preferred_element_type=jnp.float32)
      + b1_ref[...], 0.0)
  t = jnp.maximum(
      jnp.dot(t, w2_ref[...], preferred_element_type=jnp.float32)
      + b2_ref[...], 0.0)
  t = jnp.dot(t, w3_ref[...], preferred_element_type=jnp.float32) + b3_ref[...]
  ew_ref[...] = jax.nn.sigmoid(t[:, 0])[None, None, :]
  a0 = jnp.dot(ea, v0_ref[...], preferred_element_type=jnp.float32)
  a1 = jnp.dot(ea, v1_ref[...], preferred_element_type=jnp.float32)
  ae0_ref[...] = a0[:, 0][None, None, :]
  ae1_ref[...] = a1[:, 0][None, None, :]
  mx_ref[...] = jnp.concatenate([jnp.max(a0, axis=0), jnp.max(a1, axis=0)
                                 ])[None, None, :]


def _tc_edge(ea8, w1, b1, w2, b2, w3, b3, v0, v1):
  g = E // BE
  return pl.pallas_call(
      _tc_edge_body,
      grid=(g,),
      in_specs=[
          pl.BlockSpec((BE, 3), lambda i: (i, 0)),
          pl.BlockSpec((3, 32), lambda i: (0, 0)),
          pl.BlockSpec((1, 32), lambda i: (0, 0)),
          pl.BlockSpec((32, 16), lambda i: (0, 0)),
          pl.BlockSpec((1, 16), lambda i: (0, 0)),
          pl.BlockSpec((16, 1), lambda i: (0, 0)),
          pl.BlockSpec((1, 1), lambda i: (0, 0)),
          pl.BlockSpec((3, 1), lambda i: (0, 0)),
          pl.BlockSpec((3, 1), lambda i: (0, 0)),
      ],
      out_specs=[
          pl.BlockSpec((1, 1, BE), lambda i: (i, 0, 0)),
          pl.BlockSpec((1, 1, BE), lambda i: (i, 0, 0)),
          pl.BlockSpec((1, 1, BE), lambda i: (i, 0, 0)),
          pl.BlockSpec((1, 1, 2), lambda i: (i, 0, 0)),
      ],
      out_shape=[
          jax.ShapeDtypeStruct((E // BE, 1, BE), jnp.float32),
          jax.ShapeDtypeStruct((E // BE, 1, BE), jnp.float32),
          jax.ShapeDtypeStruct((E // BE, 1, BE), jnp.float32),
          jax.ShapeDtypeStruct((g, 1, 2), jnp.float32),
      ],
  )(ea8, w1, b1, w2, b2, w3, b3, v0, v1)


def _tc_mid_body(o_ref, b_ref, w_ref, as_ref, ad_ref,
                 hh_ref, asrc_ref, adst_ref, mx_ref):
  v = o_ref[...] + b_ref[...]
  h1 = jnp.where(v > 0, v, jnp.exp(jnp.minimum(v, 0.0)) - 1.0)
  hh = jnp.dot(h1, w_ref[...], preferred_element_type=jnp.float32)
  hh_ref[...] = hh
  a_s = jnp.dot(hh, as_ref[...], preferred_element_type=jnp.float32)
  a_d = jnp.dot(hh, ad_ref[...], preferred_element_type=jnp.float32)
  asrc_ref[...] = a_s[:, 0][None, None, :]
  adst_ref[...] = a_d[:, 0][None, None, :]
  mx_ref[...] = jnp.concatenate([jnp.max(a_s, axis=0), jnp.max(a_d, axis=0)
                                 ])[None, None, :]


def _tc_mid(out0, bias0, w, att_s, att_d):
  g = N // BN
  return pl.pallas_call(
      _tc_mid_body,
      grid=(g,),
      in_specs=[
          pl.BlockSpec((BN, HID), lambda i: (i, 0)),
          pl.BlockSpec((1, HID), lambda i: (0, 0)),
          pl.BlockSpec((HID, HID), lambda i: (0, 0)),
          pl.BlockSpec((HID, 1), lambda i: (0, 0)),
          pl.BlockSpec((HID, 1), lambda i: (0, 0)),
      ],
      out_specs=[
          pl.BlockSpec((BN, HID), lambda i: (i, 0)),
          pl.BlockSpec((1, 1, BN), lambda i: (i, 0, 0)),
          pl.BlockSpec((1, 1, BN), lambda i: (i, 0, 0)),
          pl.BlockSpec((1, 1, 2), lambda i: (i, 0, 0)),
      ],
      out_shape=[
          jax.ShapeDtypeStruct((N, HID), jnp.float32),
          jax.ShapeDtypeStruct((g, 1, BN), jnp.float32),
          jax.ShapeDtypeStruct((g, 1, BN), jnp.float32),
          jax.ShapeDtypeStruct((g, 1, 2), jnp.float32),
      ],
  )(out0, bias0, w, att_s, att_d)


def _tc_matvec_body(s0_ref, s1_ref, hh_ref, o_ref):
  i = pl.program_id(0)

  @pl.when(i == 0)
  def _():
    o_ref[...] = jnp.zeros_like(o_ref)
  sv = (s0_ref[0, 0, :] + s1_ref[0, 0, :])[None, :]
  o_ref[...] += jnp.dot(sv, hh_ref[...], preferred_element_type=jnp.float32)


def _tc_matvec(s2, hh1):
  g = N // BN
  s0 = s2[0].reshape(g, 1, BN)
  s1 = s2[1].reshape(g, 1, BN)
  return pl.pallas_call(
      _tc_matvec_body,
      grid=(g,),
      in_specs=[
          pl.BlockSpec((1, 1, BN), lambda i: (i, 0, 0)),
          pl.BlockSpec((1, 1, BN), lambda i: (i, 0, 0)),
          pl.BlockSpec((BN, HID), lambda i: (i, 0)),
      ],
      out_specs=pl.BlockSpec((1, HID), lambda i: (0, 0)),
      out_shape=jax.ShapeDtypeStruct((1, HID), jnp.float32),
  )(s0, s1, hh1)


def _tc_head_body(hs_ref, b1_ref, lf_ref, dp_ref,
                  r_w, r_b, c1_w, c1_b, c2_w, c2_b, c3_w, c3_b, c4_w, c4_b,
                  f1_w, f1_b, lng, lnb, f2_w, f2_b, k1_w, k1_b, k2_w, k2_b,
                  o_ref):
  hmean = hs_ref[...] * jnp.float32(1.0 / N) + b1_ref[...]
  lf = lf_ref[...].reshape(1, 64)

  def stack3(m):
    z = jnp.zeros((m.shape[0], 1), jnp.float32)
    left = jnp.concatenate([m[:, 1:], z], axis=1)
    right = jnp.concatenate([z, m[:, :-1]], axis=1)
    return jnp.concatenate([right, m, left], axis=0)

  r = jnp.dot(r_w[...], lf, preferred_element_type=jnp.float32) + r_b[...]
  a = jnp.maximum(
      jnp.dot(c1_w[...], stack3(lf), preferred_element_type=jnp.float32)
      + c1_b[...], 0.0)
  a = jnp.maximum(
      jnp.dot(c2_w[...], stack3(a), preferred_element_type=jnp.float32)
      + c2_b[...], 0.0)
  a = jnp.maximum(a + r, 0.0)
  b = jnp.maximum(
      jnp.dot(c3_w[...], stack3(a), preferred_element_type=jnp.float32)
      + c3_b[...], 0.0)
  b = jnp.maximum(
      jnp.dot(c4_w[...], stack3(b), preferred_element_type=jnp.float32)
      + c4_b[...], 0.0)
  a = jnp.maximum(b + a, 0.0)
  lfeat = jnp.mean(a, axis=0)[None, :]

  comb = jnp.concatenate([hmean, dp_ref[...], lfeat], axis=1)
  f = jnp.maximum(
      jnp.dot(comb, f1_w[...], preferred_element_type=jnp.float32)
      + f1_b[...], 0.0)
  mu = jnp.mean(f)
  var = jnp.mean((f - mu) ** 2)
  f = (f - mu) / jnp.sqrt(var + jnp.float32(1e-5)) * lng[...] + lnb[...]
  f = jnp.dot(f, f2_w[...], preferred_element_type=jnp.float32) + f2_b[...]
  c = jnp.maximum(
      jnp.dot(f, k1_w[...], preferred_element_type=jnp.float32)
      + k1_b[...], 0.0)
  o = jnp.dot(c, k2_w[...], preferred_element_type=jnp.float32) + k2_b[...]
  o = o - jnp.max(o)
  o_ref[...] = o - jnp.log(jnp.sum(jnp.exp(o)))


def _tc_head(hsum, bias1, lf, dp, tcn, head):
  full = lambda s: pl.BlockSpec(s, lambda: tuple(0 for _ in s))
  args = [hsum, bias1, lf, dp] + tcn + head
  return pl.pallas_call(
      _tc_head_body,
      in_specs=[full(tuple(a.shape)) for a in args],
      out_specs=full((1, 10)),
      out_shape=jax.ShapeDtypeStruct((1, 10), jnp.float32),
  )(*args)


def kernel(x, edge_index, edge_attr, log_features, duration, path_length,
           params):
  p = params
  f32 = jnp.float32
  src = edge_index[0].astype(jnp.int32)
  dst = edge_index[1].astype(jnp.int32)

  x8 = x.astype(f32)
  ea8 = edge_attr.astype(f32)
  enc_w1 = p['enc_w1']
  ew_w1 = p['ew_w1']
  g0, g1 = p['gat0'], p['gat1']
  v0 = g0['w_edge'] @ g0['att_edge'][0][:, None]
  v1 = g1['w_edge'] @ g1['att_edge'][0][:, None]

  zn = jnp.zeros((N,), f32)
  zacc = jnp.zeros((NHALF, HID), f32)

  hh0, asrc0, adst0, mxn = _tc_node(
      x8, enc_w1, p['enc_b1'][None, :], p['enc_w2'], p['enc_b2'][None, :],
      g0['w'], g0['att_src'][0][:, None], g0['att_dst'][0][:, None])
  asrc0, adst0 = asrc0.reshape(N), adst0.reshape(N)
  ew, ae0, ae1, mxe = _tc_edge(
      ea8, ew_w1, p['ew_b1'][None, :], p['ew_w2'], p['ew_b2'][None, :],
      p['ew_w3'], p['ew_b3'][None, :], v0, v1)
  ew, ae0, ae1 = ew.reshape(E), ae0.reshape(E), ae1.reshape(E)

  m0 = jnp.max(mxn[:, 0, 0]) + jnp.max(mxn[:, 0, 1]) + jnp.max(mxe[:, 0, 0])
  m0 = jnp.where(m0 > 0, m0, m0 * f32(0.2))
  mvec0 = jnp.full((16,), m0, f32)

  ex0, den0 = _sc_att(src, dst, ae0, asrc0, adst0, mvec0, zn)
  w0 = _sc_norm(dst, ex0, ew, den0)
  out0 = _sc_spmm(src, dst, w0, hh0, zacc)

  hh1, asrc1, adst1, mxm = _tc_mid(
      out0, g0['bias'][None, :], g1['w'],
      g1['att_src'][0][:, None], g1['att_dst'][0][:, None])
  asrc1, adst1 = asrc1.reshape(N), adst1.reshape(N)
  m1 = jnp.max(mxm[:, 0, 0]) + jnp.max(mxm[:, 0, 1]) + jnp.max(mxe[:, 0, 1])
  m1 = jnp.where(m1 > 0, m1, m1 * f32(0.2))
  mvec1 = jnp.full((16,), m1, f32)

  ex1, den1 = _sc_att(src, dst, ae1, asrc1, adst1, mvec1, zn)
  s2 = _sc_srcnorm(src, dst, ex1, ew, den1, zn)
  hsum = _tc_matvec(s2, hh1)

  dp = jnp.concatenate([duration, path_length]).astype(f32)[None, :]
  def cw(w):
    return jnp.transpose(w, (0, 2, 1)).reshape(w.shape[0], -1)
  tcn = [p['tcn_b1_dw'][:, 0, :], p['tcn_b1_db'][:, None],
         cw(p['tcn_b1_c1_w']), p['tcn_b1_c1_b'][:, None],
         cw(p['tcn_b1_c2_w']), p['tcn_b1_c2_b'][:, None],
         cw(p['tcn_b2_c1_w']), p['tcn_b2_c1_b'][:, None],
         cw(p['tcn_b2_c2_w']), p['tcn_b2_c2_b'][:, None]]
  head = [p['fus_w1'], p['fus_b1'][None, :], p['ln_g'][None, :],
          p['ln_b'][None, :], p['fus_w2'], p['fus_b2'][None, :],
          p['cls_w1'], p['cls_b1'][None, :], p['cls_w2'], p['cls_b2'][None, :]]
  out = _tc_head(hsum, g1['bias'][None, :], log_features, dp, tcn, head)
  return out[0]


# speedup vs baseline: 16.8179x; 1.0237x over previous
import functools
import jax
import jax.numpy as jnp
from jax import lax
from jax.experimental import pallas as pl
from jax.experimental.pallas import tpu as pltpu, tpu_sc as plsc

N = 50000
E = 800000
HID = 64
NC = 2
NS = 16
NW = NC * NS
NHALF = N // 2
CH = 1600
CHUNKS = E // CH
CHS = 160
CHUNKS_S = E // CHS


def _mesh():
  return plsc.VectorSubcoreMesh(core_axis_name="c", subcore_axis_name="s")


_SC_PARAMS = pltpu.CompilerParams(use_tc_tiling_on_sc=False,
                                 needs_layout_passes=False)


def _sc_att_body(src_hbm, dst_hbm, ae_hbm, asrc_hbm, adst_hbm, mvec_hbm,
                 zn_hbm, ex_hbm, den_hbm,
                 asrc_v, adst_v, src_b, dst_b, ae_b, ex_b, mv_v, sem, den_sh):
  cid = lax.axis_index("c")
  sid = lax.axis_index("s")
  wid = sid * NC + cid
  pltpu.sync_copy(asrc_hbm, asrc_v)
  pltpu.sync_copy(adst_hbm, adst_v)
  pltpu.sync_copy(mvec_hbm, mv_v)

  @pl.when(sid == 0)
  def _():
    pltpu.sync_copy(zn_hbm, den_sh)
  plsc.subcore_barrier()

  mv = mv_v[...]
  nk = (CHUNKS - wid + NW - 1) // NW

  def chunk(i, _):
    off = (wid + i * NW) * CH
    d1 = pltpu.async_copy(src_hbm.at[pl.ds(off, CH)], src_b, sem)
    d2 = pltpu.async_copy(dst_hbm.at[pl.ds(off, CH)], dst_b, sem)
    d3 = pltpu.async_copy(ae_hbm.at[pl.ds(off, CH)], ae_b, sem)
    d1.wait(); d2.wait(); d3.wait()

    def vec(j, _):
      sl = pl.ds(j * 16, 16)
      si = src_b[sl]
      di = dst_b[sl]
      a = (plsc.load_gather(asrc_v, [si]) + plsc.load_gather(adst_v, [di])
           + ae_b[sl])
      a = jnp.where(a > 0, a, a * jnp.float32(0.2))
      ex_b[sl] = jnp.exp(a - mv)
      return 0
    lax.fori_loop(0, CH // 16, vec, 0, unroll=4)
    pltpu.sync_copy(ex_b, ex_hbm.at[pl.ds(off, CH)])
    pltpu.sync_copy(ex_b, den_sh.at[dst_b], add=True)
    return 0
  lax.fori_loop(0, nk, chunk, 0)

  plsc.subcore_barrier()

  @pl.when(sid == 0)
  def _():
    pltpu.sync_copy(den_sh, den_hbm.at[cid])


def _sc_att(src, dst, ae, asrc, adst, mvec, zn):
  k = pl.kernel(
      _sc_att_body,
      out_type=[jax.ShapeDtypeStruct((E,), jnp.float32),
                jax.ShapeDtypeStruct((NC, N), jnp.float32)],
      mesh=_mesh(),
      compiler_params=_SC_PARAMS,
      scratch_types=[
          pltpu.VMEM((N,), jnp.float32),
          pltpu.VMEM((N,), jnp.float32),
          pltpu.VMEM((CH,), jnp.int32),
          pltpu.VMEM((CH,), jnp.int32),
          pltpu.VMEM((CH,), jnp.float32),
          pltpu.VMEM((CH,), jnp.float32),
          pltpu.VMEM((16,), jnp.float32),
          pltpu.SemaphoreType.DMA,
          pltpu.VMEM_SHARED((N,), jnp.float32),
      ],
  )
  return k(src, dst, ae, asrc, adst, mvec, zn)


def _sc_norm_body(dst_hbm, ex_hbm, ew_hbm, den_hbm,
                  w_hbm,
                  den0_v, den1_v, dst_b, ex_b, ew_b, w_b, sem):
  cid = lax.axis_index("c")
  sid = lax.axis_index("s")
  wid = sid * NC + cid
  pltpu.sync_copy(den_hbm.at[0], den0_v)
  pltpu.sync_copy(den_hbm.at[1], den1_v)
  nk = (CHUNKS - wid + NW - 1) // NW

  def chunk(i, _):
    off = (wid + i * NW) * CH
    d1 = pltpu.async_copy(dst_hbm.at[pl.ds(off, CH)], dst_b, sem)
    d2 = pltpu.async_copy(ex_hbm.at[pl.ds(off, CH)], ex_b, sem)
    d3 = pltpu.async_copy(ew_hbm.at[pl.ds(off, CH)], ew_b, sem)
    d1.wait(); d2.wait(); d3.wait()

    def vec(j, _):
      sl = pl.ds(j * 16, 16)
      di = dst_b[sl]
      den = plsc.load_gather(den0_v, [di]) + plsc.load_gather(den1_v, [di])
      w_b[sl] = ex_b[sl] / (den + jnp.float32(1e-16)) * ew_b[sl]
      return 0
    lax.fori_loop(0, CH // 16, vec, 0, unroll=4)
    pltpu.sync_copy(w_b, w_hbm.at[pl.ds(off, CH)])
    return 0
  lax.fori_loop(0, nk, chunk, 0)


def _sc_norm(dst, ex, ew, den2):
  k = pl.kernel(
      _sc_norm_body,
      out_type=[jax.ShapeDtypeStruct((E,), jnp.float32)],
      mesh=_mesh(),
      compiler_params=_SC_PARAMS,
      scratch_types=[
          pltpu.VMEM((N,), jnp.float32),
          pltpu.VMEM((N,), jnp.float32),
          pltpu.VMEM((CH,), jnp.int32),
          pltpu.VMEM((CH,), jnp.float32),
          pltpu.VMEM((CH,), jnp.float32),
          pltpu.VMEM((CH,), jnp.float32),
          pltpu.SemaphoreType.DMA,
      ],
  )
  return k(dst, ex, ew, den2)[0]


def _sc_spmm_body(src_hbm, dst_hbm, w_hbm, hh_hbm, zacc_hbm,
                  out_hbm,
                  src_a, dst_a, w_a, wm_a, dr_a, rows_a, sem_a,
                  src_c, dst_c, w_c, wm_c, dr_c, rows_c, sem_c, acc_sh):
  cid = lax.axis_index("c")
  sid = lax.axis_index("s")
  lo = cid * NHALF

  @pl.when(sid == 0)
  def _():
    pltpu.sync_copy(zacc_hbm, acc_sh)
  plsc.subcore_barrier()

  nk = (CHUNKS_S - sid + NS - 1) // NS

  def load_and_fire(c, bufs):
    src_b, dst_b, w_b, wm_b, dr_b, rows, sem = bufs
    off = (sid + c * NS) * CHS
    d1 = pltpu.async_copy(src_hbm.at[pl.ds(off, CHS)], src_b, sem)
    d2 = pltpu.async_copy(dst_hbm.at[pl.ds(off, CHS)], dst_b, sem)
    d3 = pltpu.async_copy(w_hbm.at[pl.ds(off, CHS)], w_b, sem)
    d1.wait(); d2.wait(); d3.wait()

    def vec(j, _):
      sl = pl.ds(j * 16, 16)
      di = dst_b[sl]
      m = (di >= lo) & (di < lo + NHALF)
      wm_b[sl] = jnp.where(m, w_b[sl], jnp.float32(0.0))
      dr_b[sl] = jnp.where(m, di - lo, 0)
      return 0
    lax.fori_loop(0, CHS // 16, vec, 0, unroll=4)
    return pltpu.async_copy(hh_hbm.at[src_b], rows, sem)

  def drain(desc, bufs):
    _, _, _, wm_b, dr_b, rows, _ = bufs
    desc.wait()

    def scale(j, _):
      wv = wm_b[pl.ds(j * 16, 16)]
      for r in range(16):
        ws = wv[r]
        row = j * 16 + r
        for q in range(4):
          cs = pl.ds(q * 16, 16)
          rows[row, cs] = rows[row, cs] * ws
      return 0
    lax.fori_loop(0, CHS // 16, scale, 0)
    pltpu.sync_copy(rows, acc_sh.at[dr_b], add=True)

  bufs_a = (src_a, dst_a, w_a, wm_a, dr_a, rows_a, sem_a)
  bufs_c = (src_c, dst_c, w_c, wm_c, dr_c, rows_c, sem_c)

  def pair(t, _):
    d0 = load_and_fire(2 * t, bufs_a)
    d1 = load_and_fire(2 * t + 1, bufs_c)
    drain(d0, bufs_a)
    drain(d1, bufs_c)
    return 0
  lax.fori_loop(0, nk // 2, pair, 0)

  @pl.when(nk % 2 == 1)
  def _():
    drain(load_and_fire(nk - 1, bufs_a), bufs_a)

  plsc.subcore_barrier()

  @pl.when(sid == 0)
  def _():
    pltpu.sync_copy(acc_sh, out_hbm.at[pl.ds(lo, NHALF)])


def _sc_spmm(src, dst, w, hh0, zacc):
  buf = lambda: [pltpu.VMEM((CHS,), jnp.int32),
                 pltpu.VMEM((CHS,), jnp.int32),
                 pltpu.VMEM((CHS,), jnp.float32),
                 pltpu.VMEM((CHS,), jnp.float32),
                 pltpu.VMEM((CHS,), jnp.int32),
                 pltpu.VMEM((CHS, HID), jnp.float32),
                 pltpu.SemaphoreType.DMA]
  k = pl.kernel(
      _sc_spmm_body,
      out_type=[jax.ShapeDtypeStruct((N, HID), jnp.float32)],
      mesh=_mesh(),
      compiler_params=_SC_PARAMS,
      scratch_types=buf() + buf() + [
          pltpu.VMEM_SHARED((NHALF, HID), jnp.float32),
      ],
  )
  return k(src, dst, w, hh0, zacc)[0]


def _sc_srcnorm_body(src_hbm, dst_hbm, ex_hbm, ew_hbm, den_hbm, zn_hbm,
                     s_hbm,
                     den0_v, den1_v, src_b, dst_b, ex_b, ew_b, w_b, sem, s_sh):
  cid = lax.axis_index("c")
  sid = lax.axis_index("s")
  wid = sid * NC + cid
  pltpu.sync_copy(den_hbm.at[0], den0_v)
  pltpu.sync_copy(den_hbm.at[1], den1_v)

  @pl.when(sid == 0)
  def _():
    pltpu.sync_copy(zn_hbm, s_sh)
  plsc.subcore_barrier()

  nk = (CHUNKS - wid + NW - 1) // NW

  def chunk(i, _):
    off = (wid + i * NW) * CH
    d1 = pltpu.async_copy(src_hbm.at[pl.ds(off, CH)], src_b, sem)
    d2 = pltpu.async_copy(dst_hbm.at[pl.ds(off, CH)], dst_b, sem)
    d3 = pltpu.async_copy(ex_hbm.at[pl.ds(off, CH)], ex_b, sem)
    d4 = pltpu.async_copy(ew_hbm.at[pl.ds(off, CH)], ew_b, sem)
    d1.wait(); d2.wait(); d3.wait(); d4.wait()

    def vec(j, _):
      sl = pl.ds(j * 16, 16)
      di = dst_b[sl]
      den = plsc.load_gather(den0_v, [di]) + plsc.load_gather(den1_v, [di])
      w_b[sl] = ex_b[sl] / (den + jnp.float32(1e-16)) * ew_b[sl]
      return 0
    lax.fori_loop(0, CH // 16, vec, 0, unroll=4)
    pltpu.sync_copy(w_b, s_sh.at[src_b], add=True)
    return 0
  lax.fori_loop(0, nk, chunk, 0)

  plsc.subcore_barrier()

  @pl.when(sid == 0)
  def _():
    pltpu.sync_copy(s_sh, s_hbm.at[cid])


def _sc_srcnorm(src, dst, ex, ew, den2, zn):
  k = pl.kernel(
      _sc_srcnorm_body,
      out_type=[jax.ShapeDtypeStruct((NC, N), jnp.float32)],
      mesh=_mesh(),
      compiler_params=_SC_PARAMS,
      scratch_types=[
          pltpu.VMEM((N,), jnp.float32),
          pltpu.VMEM((N,), jnp.float32),
          pltpu.VMEM((CH,), jnp.int32),
          pltpu.VMEM((CH,), jnp.int32),
          pltpu.VMEM((CH,), jnp.float32),
          pltpu.VMEM((CH,), jnp.float32),
          pltpu.VMEM((CH,), jnp.float32),
          pltpu.SemaphoreType.DMA,
          pltpu.VMEM_SHARED((N,), jnp.float32),
      ],
  )
  return k(src, dst, ex, ew, den2, zn)[0]


BN = 2000
BE = 8000


def _tc_node_body(x_ref, w1_ref, b1_ref, w2_ref, b2_ref, w_ref, as_ref,
                  ad_ref, hh_ref, asrc_ref, adst_ref, mx_ref):
  h = jnp.maximum(
      jnp.dot(x_ref[...], w1_ref[...], preferred_element_type=jnp.float32)
      + b1_ref[...], 0.0)
  h = jnp.dot(h, w2_ref[...], preferred_element_type=jnp.float32) + b2_ref[...]
  hh = jnp.dot(h, w_ref[...], preferred_element_type=jnp.float32)
  hh_ref[...] = hh
  a_s = jnp.dot(hh, as_ref[...], preferred_element_type=jnp.float32)
  a_d = jnp.dot(hh, ad_ref[...], preferred_element_type=jnp.float32)
  asrc_ref[...] = a_s[:, 0][None, None, :]
  adst_ref[...] = a_d[:, 0][None, None, :]
  mx_ref[...] = jnp.concatenate([jnp.max(a_s, axis=0), jnp.max(a_d, axis=0)
                                 ])[None, None, :]


def _tc_node(x8, w1, b1, w2, b2, w, att_s, att_d):
  g = N // BN
  return pl.pallas_call(
      _tc_node_body,
      grid=(g,),
      in_specs=[
          pl.BlockSpec((BN, 5), lambda i: (i, 0)),
          pl.BlockSpec((5, 32), lambda i: (0, 0)),
          pl.BlockSpec((1, 32), lambda i: (0, 0)),
          pl.BlockSpec((32, HID), lambda i: (0, 0)),
          pl.BlockSpec((1, HID), lambda i: (0, 0)),
          pl.BlockSpec((HID, HID), lambda i: (0, 0)),
          pl.BlockSpec((HID, 1), lambda i: (0, 0)),
          pl.BlockSpec((HID, 1), lambda i: (0, 0)),
      ],
      out_specs=[
          pl.BlockSpec((BN, HID), lambda i: (i, 0)),
          pl.BlockSpec((1, 1, BN), lambda i: (i, 0, 0)),
          pl.BlockSpec((1, 1, BN), lambda i: (i, 0, 0)),
          pl.BlockSpec((1, 1, 2), lambda i: (i, 0, 0)),
      ],
      out_shape=[
          jax.ShapeDtypeStruct((N, HID), jnp.float32),
          jax.ShapeDtypeStruct((g, 1, BN), jnp.float32),
          jax.ShapeDtypeStruct((g, 1, BN), jnp.float32),
          jax.ShapeDtypeStruct((g, 1, 2), jnp.float32),
      ],
  )(x8, w1, b1, w2, b2, w, att_s, att_d)


def _tc_edge_body(ea_ref, w1_ref, b1_ref, w2_ref, b2_ref, w3_ref, b3_ref,
                  v0_ref, v1_ref, ew_ref, ae0_ref, ae1_ref, mx_ref):
  ea = ea_ref[...]
  t = jnp.maximum(
      jnp.dot(ea, w1_ref[...], preferred_element_type=jnp.float32)
      + b1_ref[...], 0.0)
  t = jnp.maximum(
      jnp.dot(t, w2_ref[...], preferred_element_type=jnp.float32)
      + b2_ref[...], 0.0)
  t = jnp.dot(t, w3_ref[...], preferred_element_type=jnp.float32) + b3_ref[...]
  ew_ref[...] = jax.nn.sigmoid(t[:, 0])[None, None, :]
  a0 = jnp.dot(ea, v0_ref[...], preferred_element_type=jnp.float32)
  a1 = jnp.dot(ea, v1_ref[...], preferred_element_type=jnp.float32)
  ae0_ref[...] = a0[:, 0][None, None, :]
  ae1_ref[...] = a1[:, 0][None, None, :]
  mx_ref[...] = jnp.concatenate([jnp.max(a0, axis=0), jnp.max(a1, axis=0)
                                 ])[None, None, :]


def _tc_edge(ea8, w1, b1, w2, b2, w3, b3, v0, v1):
  g = E // BE
  return pl.pallas_call(
      _tc_edge_body,
      grid=(g,),
      in_specs=[
          pl.BlockSpec((BE, 3), lambda i: (i, 0)),
          pl.BlockSpec((3, 32), lambda i: (0, 0)),
          pl.BlockSpec((1, 32), lambda i: (0, 0)),
          pl.BlockSpec((32, 16), lambda i: (0, 0)),
          pl.BlockSpec((1, 16), lambda i: (0, 0)),
          pl.BlockSpec((16, 1), lambda i: (0, 0)),
          pl.BlockSpec((1, 1), lambda i: (0, 0)),
          pl.BlockSpec((3, 1), lambda i: (0, 0)),
          pl.BlockSpec((3, 1), lambda i: (0, 0)),
      ],
      out_specs=[
          pl.BlockSpec((1, 1, BE), lambda i: (i, 0, 0)),
          pl.BlockSpec((1, 1, BE), lambda i: (i, 0, 0)),
          pl.BlockSpec((1, 1, BE), lambda i: (i, 0, 0)),
          pl.BlockSpec((1, 1, 2), lambda i: (i, 0, 0)),
      ],
      out_shape=[
          jax.ShapeDtypeStruct((E // BE, 1, BE), jnp.float32),
          jax.ShapeDtypeStruct((E // BE, 1, BE), jnp.float32),
          jax.ShapeDtypeStruct((E // BE, 1, BE), jnp.float32),
          jax.ShapeDtypeStruct((g, 1, 2), jnp.float32),
      ],
  )(ea8, w1, b1, w2, b2, w3, b3, v0, v1)


def _tc_mid_body(o_ref, b_ref, w_ref, as_ref, ad_ref,
                 hh_ref, asrc_ref, adst_ref, mx_ref):
  v = o_ref[...] + b_ref[...]
  h1 = jnp.where(v > 0, v, jnp.exp(jnp.minimum(v, 0.0)) - 1.0)
  hh = jnp.dot(h1, w_ref[...], preferred_element_type=jnp.float32)
  hh_ref[...] = hh
  a_s = jnp.dot(hh, as_ref[...], preferred_element_type=jnp.float32)
  a_d = jnp.dot(hh, ad_ref[...], preferred_element_type=jnp.float32)
  asrc_ref[...] = a_s[:, 0][None, None, :]
  adst_ref[...] = a_d[:, 0][None, None, :]
  mx_ref[...] = jnp.concatenate([jnp.max(a_s, axis=0), jnp.max(a_d, axis=0)
                                 ])[None, None, :]


def _tc_mid(out0, bias0, w, att_s, att_d):
  g = N // BN
  return pl.pallas_call(
      _tc_mid_body,
      grid=(g,),
      in_specs=[
          pl.BlockSpec((BN, HID), lambda i: (i, 0)),
          pl.BlockSpec((1, HID), lambda i: (0, 0)),
          pl.BlockSpec((HID, HID), lambda i: (0, 0)),
          pl.BlockSpec((HID, 1), lambda i: (0, 0)),
          pl.BlockSpec((HID, 1), lambda i: (0, 0)),
      ],
      out_specs=[
          pl.BlockSpec((BN, HID), lambda i: (i, 0)),
          pl.BlockSpec((1, 1, BN), lambda i: (i, 0, 0)),
          pl.BlockSpec((1, 1, BN), lambda i: (i, 0, 0)),
          pl.BlockSpec((1, 1, 2), lambda i: (i, 0, 0)),
      ],
      out_shape=[
          jax.ShapeDtypeStruct((N, HID), jnp.float32),
          jax.ShapeDtypeStruct((g, 1, BN), jnp.float32),
          jax.ShapeDtypeStruct((g, 1, BN), jnp.float32),
          jax.ShapeDtypeStruct((g, 1, 2), jnp.float32),
      ],
  )(out0, bias0, w, att_s, att_d)


def _tc_matvec_body(s0_ref, s1_ref, hh_ref, o_ref):
  i = pl.program_id(0)

  @pl.when(i == 0)
  def _():
    o_ref[...] = jnp.zeros_like(o_ref)
  sv = (s0_ref[0, 0, :] + s1_ref[0, 0, :])[None, :]
  o_ref[...] += jnp.dot(sv, hh_ref[...], preferred_element_type=jnp.float32)


def _tc_matvec(s2, hh1):
  g = N // BN
  s0 = s2[0].reshape(g, 1, BN)
  s1 = s2[1].reshape(g, 1, BN)
  return pl.pallas_call(
      _tc_matvec_body,
      grid=(g,),
      in_specs=[
          pl.BlockSpec((1, 1, BN), lambda i: (i, 0, 0)),
          pl.BlockSpec((1, 1, BN), lambda i: (i, 0, 0)),
          pl.BlockSpec((BN, HID), lambda i: (i, 0)),
      ],
      out_specs=pl.BlockSpec((1, HID), lambda i: (0, 0)),
      out_shape=jax.ShapeDtypeStruct((1, HID), jnp.float32),
  )(s0, s1, hh1)


def _tc_head_body(hs_ref, b1_ref, lf_ref, dp_ref,
                  r_w, r_b, c1_w, c1_b, c2_w, c2_b, c3_w, c3_b, c4_w, c4_b,
                  f1_w, f1_b, lng, lnb, f2_w, f2_b, k1_w, k1_b, k2_w, k2_b,
                  o_ref):
  hmean = hs_ref[...] * jnp.float32(1.0 / N) + b1_ref[...]
  lf = lf_ref[...].reshape(1, 64)

  def stack3(m):
    z = jnp.zeros((m.shape[0], 1), jnp.float32)
    left = jnp.concatenate([m[:, 1:], z], axis=1)
    right = jnp.concatenate([z, m[:, :-1]], axis=1)
    return jnp.concatenate([right, m, left], axis=0)

  r = jnp.dot(r_w[...], lf, preferred_element_type=jnp.float32) + r_b[...]
  a = jnp.maximum(
      jnp.dot(c1_w[...], stack3(lf), preferred_element_type=jnp.float32)
      + c1_b[...], 0.0)
  a = jnp.maximum(
      jnp.dot(c2_w[...], stack3(a), preferred_element_type=jnp.float32)
      + c2_b[...], 0.0)
  a = jnp.maximum(a + r, 0.0)
  b = jnp.maximum(
      jnp.dot(c3_w[...], stack3(a), preferred_element_type=jnp.float32)
      + c3_b[...], 0.0)
  b = jnp.maximum(
      jnp.dot(c4_w[...], stack3(b), preferred_element_type=jnp.float32)
      + c4_b[...], 0.0)
  a = jnp.maximum(b + a, 0.0)
  lfeat = jnp.mean(a, axis=0)[None, :]

  comb = jnp.concatenate([hmean, dp_ref[...], lfeat], axis=1)
  f = jnp.maximum(
      jnp.dot(comb, f1_w[...], preferred_element_type=jnp.float32)
      + f1_b[...], 0.0)
  mu = jnp.mean(f)
  var = jnp.mean((f - mu) ** 2)
  f = (f - mu) / jnp.sqrt(var + jnp.float32(1e-5)) * lng[...] + lnb[...]
  f = jnp.dot(f, f2_w[...], preferred_element_type=jnp.float32) + f2_b[...]
  c = jnp.maximum(
      jnp.dot(f, k1_w[...], preferred_element_type=jnp.float32)
      + k1_b[...], 0.0)
  o = jnp.dot(c, k2_w[...], preferred_element_type=jnp.float32) + k2_b[...]
  o = o - jnp.max(o)
  o_ref[...] = o - jnp.log(jnp.sum(jnp.exp(o)))


def _tc_head(hsum, bias1, lf, dp, tcn, head):
  full = lambda s: pl.BlockSpec(s, lambda: tuple(0 for _ in s))
  args = [hsum, bias1, lf, dp] + tcn + head
  return pl.pallas_call(
      _tc_head_body,
      in_specs=[full(tuple(a.shape)) for a in args],
      out_specs=full((1, 10)),
      out_shape=jax.ShapeDtypeStruct((1, 10), jnp.float32),
  )(*args)


def kernel(x, edge_index, edge_attr, log_features, duration, path_length,
           params):
  p = params
  f32 = jnp.float32
  src = edge_index[0].astype(jnp.int32)
  dst = edge_index[1].astype(jnp.int32)

  x8 = x.astype(f32)
  ea8 = edge_attr.astype(f32)
  enc_w1 = p['enc_w1']
  ew_w1 = p['ew_w1']
  g0, g1 = p['gat0'], p['gat1']
  v0 = g0['w_edge'] @ g0['att_edge'][0][:, None]
  v1 = g1['w_edge'] @ g1['att_edge'][0][:, None]

  zn = jnp.zeros((N,), f32)
  zacc = jnp.zeros((NHALF, HID), f32)

  hh0, asrc0, adst0, mxn = _tc_node(
      x8, enc_w1, p['enc_b1'][None, :], p['enc_w2'], p['enc_b2'][None, :],
      g0['w'], g0['att_src'][0][:, None], g0['att_dst'][0][:, None])
  asrc0, adst0 = asrc0.reshape(N), adst0.reshape(N)
  ew, ae0, ae1, mxe = _tc_edge(
      ea8, ew_w1, p['ew_b1'][None, :], p['ew_w2'], p['ew_b2'][None, :],
      p['ew_w3'], p['ew_b3'][None, :], v0, v1)
  ew, ae0, ae1 = ew.reshape(E), ae0.reshape(E), ae1.reshape(E)

  m0 = jnp.max(mxn[:, 0, 0]) + jnp.max(mxn[:, 0, 1]) + jnp.max(mxe[:, 0, 0])
  m0 = jnp.where(m0 > 0, m0, m0 * f32(0.2))
  mvec0 = jnp.full((16,), m0, f32)

  ex0, den0 = _sc_att(src, dst, ae0, asrc0, adst0, mvec0, zn)
  w0 = _sc_norm(dst, ex0, ew, den0)
  out0 = _sc_spmm(src, dst, w0, hh0, zacc)

  hh1, asrc1, adst1, mxm = _tc_mid(
      out0, g0['bias'][None, :], g1['w'],
      g1['att_src'][0][:, None], g1['att_dst'][0][:, None])
  asrc1, adst1 = asrc1.reshape(N), adst1.reshape(N)
  m1 = jnp.max(mxm[:, 0, 0]) + jnp.max(mxm[:, 0, 1]) + jnp.max(mxe[:, 0, 1])
  m1 = jnp.where(m1 > 0, m1, m1 * f32(0.2))
  mvec1 = jnp.full((16,), m1, f32)

  ex1, den1 = _sc_att(src, dst, ae1, asrc1, adst1, mvec1, zn)
  s2 = _sc_srcnorm(src, dst, ex1, ew, den1, zn)
  hsum = _tc_matvec(s2, hh1)

  dp = jnp.concatenate([duration, path_length]).astype(f32)[None, :]
  def cw(w):
    return jnp.transpose(w, (0, 2, 1)).reshape(w.shape[0], -1)
  tcn = [p['tcn_b1_dw'][:, 0, :], p['tcn_b1_db'][:, None],
         cw(p['tcn_b1_c1_w']), p['tcn_b1_c1_b'][:, None],
         cw(p['tcn_b1_c2_w']), p['tcn_b1_c2_b'][:, None],
         cw(p['tcn_b2_c1_w']), p['tcn_b2_c1_b'][:, None],
         cw(p['tcn_b2_c2_w']), p['tcn_b2_c2_b'][:, None]]
  head = [p['fus_w1'], p['fus_b1'][None, :], p['ln_g'][None, :],
          p['ln_b'][None, :], p['fus_w2'], p['fus_b2'][None, :],
          p['cls_w1'], p['cls_b1'][None, :], p['cls_w2'], p['cls_b2'][None, :]]
  out = _tc_head(hsum, g1['bias'][None, :], log_features, dp, tcn, head)
  return out[0]
